# baseline (speedup 1.0000x reference)
import functools

import jax
import jax.numpy as jnp
from jax import lax
from jax.experimental import pallas as pl
from jax.experimental.pallas import tpu as pltpu
from jax.experimental.pallas import tpu_sc as plsc

N_NODES = 10000
D_IN = 128
D_HID = 256
D_OUT = 128

NC = 2
NS = 16
NW = NC * NS
G = 128
N_PAD = 10240
ROWS_PER_TILE = N_PAD // NS
DUMMY = N_NODES
BN = 512
ZR = 16


def _fill_rows(ref, nrows, ncols, val):
    def body(i, _):
        for k in range(ncols // 16):
            ref[i, pl.ds(k * 16, 16)] = jnp.full((16,), val, jnp.float32)
        return 0
    lax.fori_loop(0, nrows, body, 0)


def _fill_ones_slot(rows_v):
    def body(i, _):
        for k in range(D_IN // 16):
            rows_v[0, i, pl.ds(k * 16, 16)] = jnp.ones((16,), jnp.float32)
        return 0
    lax.fori_loop(0, G, body, 0)


def _zero_slice(zbuf, dst_sh, base, nrows, sem):
    def issue(k, _):
        pltpu.async_copy(zbuf, dst_sh.at[pl.ds(base + k * ZR, ZR)], sem)
        return 0
    lax.fori_loop(0, nrows // ZR, issue, 0)

    def drain(k, _):
        pltpu.make_async_copy(zbuf, dst_sh.at[pl.ds(base, ZR)], sem).wait()
        return 0
    lax.fori_loop(0, nrows // ZR, drain, 0)


def _load_idx(hbm2d, idx_v, row0, ng):
    def body(k, _):
        pltpu.sync_copy(hbm2d.at[pl.ds(row0 + k * 8, 8)], idx_v.at[pl.ds(k * 8, 8)])
        return 0
    lax.fori_loop(0, ng // 8, body, 0)


def _pipelined_agg(tab, src_v, dst_v, rows_v, acc_sh, gs0, gs1, ss0, ss1, ng):
    r0 = rows_v.at[0]
    r1 = rows_v.at[1]
    pltpu.async_copy(tab.at[src_v.at[0]], r0, gs0)
    pltpu.async_copy(tab.at[src_v.at[1]], r1, gs1)

    def pair(p, _):
        g0 = 2 * p
        g1 = g0 + 1
        n0 = jnp.minimum(g0 + 2, ng - 1)
        n1 = jnp.minimum(g1 + 2, ng - 1)
        pltpu.make_async_copy(tab.at[src_v.at[g0]], r0, gs0).wait()
        pltpu.async_copy(r0, acc_sh.at[dst_v.at[g0]], ss0, add=True)
        pltpu.make_async_copy(r0, acc_sh.at[dst_v.at[g0]], ss0).wait()
        pltpu.async_copy(tab.at[src_v.at[n0]], r0, gs0)
        pltpu.make_async_copy(tab.at[src_v.at[g1]], r1, gs1).wait()
        pltpu.async_copy(r1, acc_sh.at[dst_v.at[g1]], ss1, add=True)
        pltpu.make_async_copy(r1, acc_sh.at[dst_v.at[g1]], ss1).wait()
        pltpu.async_copy(tab.at[src_v.at[n1]], r1, gs1)
        return 0

    lax.fori_loop(0, ng // 2, pair, 0)
    pltpu.make_async_copy(tab.at[src_v.at[0]], r0, gs0).wait()
    pltpu.make_async_copy(tab.at[src_v.at[0]], r1, gs1).wait()


def _deg_scatter(rows_v, dst_v, acc_sh, ss0, ng):
    r0 = rows_v.at[0]

    def chunk(cc, _):
        for j in range(8):
            pltpu.async_copy(r0, acc_sh.at[dst_v.at[cc * 8 + j]], ss0, add=True)
        for j in range(8):
            pltpu.make_async_copy(r0, acc_sh.at[dst_v.at[cc * 8 + j]], ss0).wait()
        return 0

    lax.fori_loop(0, ng // 8, chunk, 0)


def _make_sc_agg0(ng0, ng1):
    mesh = plsc.VectorSubcoreMesh(
        core_axis_name="c", subcore_axis_name="s", num_cores=NC, num_subcores=NS
    )

    @functools.partial(
        pl.kernel,
        out_type=(
            jax.ShapeDtypeStruct((2, NC, N_PAD, D_IN), jnp.float32),
            jax.ShapeDtypeStruct((2, NC, N_PAD, D_IN), jnp.float32),
        ),
        mesh=mesh,
        scratch_types=(
            pltpu.VMEM_SHARED((N_PAD, D_IN), jnp.float32),
            pltpu.VMEM((max(ng0, ng1), G), jnp.int32),
            pltpu.VMEM((max(ng0, ng1), G), jnp.int32),
            pltpu.VMEM((2, G, D_IN), jnp.float32),
            pltpu.VMEM((ZR, D_IN), jnp.float32),
            pltpu.SemaphoreType.DMA,
            pltpu.SemaphoreType.DMA,
            pltpu.SemaphoreType.DMA,
            pltpu.SemaphoreType.DMA,
        ),
    )
    def sc_agg0(x_hbm, s0_hbm, d0_hbm, s1_hbm, d1_hbm, agg_out, deg_out,
                acc_sh, src_v, dst_v, rows_v, zbuf, gs0, gs1, ss0, ss1):
        c = lax.axis_index("c")
        s = lax.axis_index("s")
        ng = jnp.where(c == 0, ng0, ng1)
        row0 = jnp.where(c == 0, s * ng0, NS * ng0 + s * ng1)
        base = s * ROWS_PER_TILE

        _fill_rows(zbuf, ZR, D_IN, 0.0)
        _zero_slice(zbuf, acc_sh, base, ROWS_PER_TILE, ss0)
        plsc.subcore_barrier()

        for r, (sh, dh) in enumerate(((s0_hbm, d0_hbm), (s1_hbm, d1_hbm))):
            _load_idx(sh, src_v, row0, ng)
            _load_idx(dh, dst_v, row0, ng)
            _pipelined_agg(x_hbm, src_v, dst_v, rows_v, acc_sh,
                           gs0, gs1, ss0, ss1, ng)
            plsc.subcore_barrier()
            pltpu.sync_copy(acc_sh.at[pl.ds(base, ROWS_PER_TILE)],
                            agg_out.at[r, c, pl.ds(base, ROWS_PER_TILE)])
            _zero_slice(zbuf, acc_sh, base, ROWS_PER_TILE, ss0)
            plsc.subcore_barrier()

        _fill_ones_slot(rows_v)
        for r, dh in enumerate((d0_hbm, d1_hbm)):
            _load_idx(dh, dst_v, row0, ng)
            _deg_scatter(rows_v, dst_v, acc_sh, ss0, ng)
            plsc.subcore_barrier()
            pltpu.sync_copy(acc_sh.at[pl.ds(base, ROWS_PER_TILE)],
                            deg_out.at[r, c, pl.ds(base, ROWS_PER_TILE)])
            if r == 0:
                _zero_slice(zbuf, acc_sh, base, ROWS_PER_TILE, ss0)
                plsc.subcore_barrier()

    return sc_agg0


def _make_sc_agg1(ng0, ng1):
    mesh = plsc.VectorSubcoreMesh(
        core_axis_name="c", subcore_axis_name="s", num_cores=NC, num_subcores=NS
    )

    @functools.partial(
        pl.kernel,
        out_type=jax.ShapeDtypeStruct((2, 2, NC, N_PAD, D_IN), jnp.float32),
        mesh=mesh,
        scratch_types=(
            pltpu.VMEM_SHARED((N_PAD, D_IN), jnp.float32),
            pltpu.VMEM((max(ng0, ng1), G), jnp.int32),
            pltpu.VMEM((max(ng0, ng1), G), jnp.int32),
            pltpu.VMEM((2, G, D_IN), jnp.float32),
            pltpu.VMEM((ZR, D_IN), jnp.float32),
            pltpu.SemaphoreType.DMA,
            pltpu.SemaphoreType.DMA,
            pltpu.SemaphoreType.DMA,
            pltpu.SemaphoreType.DMA,
        ),
    )
    def sc_agg1(h0_hbm, h1_hbm, s0_hbm, d0_hbm, s1_hbm, d1_hbm, agg_out,
                acc_sh, src_v, dst_v, rows_v, zbuf, gs0, gs1, ss0, ss1):
        c = lax.axis_index("c")
        s = lax.axis_index("s")
        ng = jnp.where(c == 0, ng0, ng1)
        row0 = jnp.where(c == 0, s * ng0, NS * ng0 + s * ng1)
        base = s * ROWS_PER_TILE

        _fill_rows(zbuf, ZR, D_IN, 0.0)
        _zero_slice(zbuf, acc_sh, base, ROWS_PER_TILE, ss0)
        plsc.subcore_barrier()

        phases = (
            (0, 0, h0_hbm, s0_hbm, d0_hbm),
            (0, 1, h1_hbm, s0_hbm, d0_hbm),
            (1, 0, h0_hbm, s1_hbm, d1_hbm),
            (1, 1, h1_hbm, s1_hbm, d1_hbm),
        )
        for i, (r, half, tab, sh, dh) in enumerate(phases):
            _load_idx(sh, src_v, row0, ng)
            _load_idx(dh, dst_v, row0, ng)
            _pipelined_agg(tab, src_v, dst_v, rows_v, acc_sh,
                           gs0, gs1, ss0, ss1, ng)
            plsc.subcore_barrier()
            pltpu.sync_copy(acc_sh.at[pl.ds(base, ROWS_PER_TILE)],
                            agg_out.at[r, half, c, pl.ds(base, ROWS_PER_TILE)])
            if i < 3:
                _zero_slice(zbuf, acc_sh, base, ROWS_PER_TILE, ss0)
                plsc.subcore_barrier()

    return sc_agg1


def _norms(deg):
    n0 = 1.0 / jnp.clip(deg[0, 0, :, 0] + deg[0, 1, :, 0], 1.0, None)
    n1 = 1.0 / jnp.clip(deg[1, 0, :, 0] + deg[1, 1, :, 0], 1.0, None)
    return n0, n1


def _tc1_body(agg_ref, deg_ref, w00_ref, w01_ref, b00_ref, b01_ref,
              h0_ref, h1_ref):
    agg = agg_ref[...]
    n0, n1 = _norms(deg_ref[...])
    a0 = (agg[0, 0] + agg[0, 1]) * n0[:, None]
    a1 = (agg[1, 0] + agg[1, 1]) * n1[:, None]
    h = jnp.dot(a0, w00_ref[...], preferred_element_type=jnp.float32)
    h += jnp.dot(a1, w01_ref[...], preferred_element_type=jnp.float32)
    h += b00_ref[...] + b01_ref[...]
    h = jnp.maximum(h, 0.0)
    h0_ref[...] = h[:, :D_IN]
    h1_ref[...] = h[:, D_IN:]


def _tc2_body(agg_ref, deg_ref, w10_ref, w11_ref, b10_ref, b11_ref, out_ref):
    agg = agg_ref[...]
    n0, n1 = _norms(deg_ref[...])
    w10 = w10_ref[...]
    w11 = w11_ref[...]
    acc = jnp.dot((agg[0, 0, 0] + agg[0, 0, 1]) * n0[:, None], w10[:D_IN],
                  preferred_element_type=jnp.float32)
    acc += jnp.dot((agg[0, 1, 0] + agg[0, 1, 1]) * n0[:, None], w10[D_IN:],
                   preferred_element_type=jnp.float32)
    acc += jnp.dot((agg[1, 0, 0] + agg[1, 0, 1]) * n1[:, None], w11[:D_IN],
                   preferred_element_type=jnp.float32)
    acc += jnp.dot((agg[1, 1, 0] + agg[1, 1, 1]) * n1[:, None], w11[D_IN:],
                   preferred_element_type=jnp.float32)
    out_ref[...] = acc + b10_ref[...] + b11_ref[...]


def _tc1(agg0, deg, w00, w01, b00, b01):
    grid = (N_PAD // BN,)
    return pl.pallas_call(
        _tc1_body,
        grid=grid,
        in_specs=[
            pl.BlockSpec((2, NC, BN, D_IN), lambda i: (0, 0, i, 0)),
            pl.BlockSpec((2, NC, BN, D_IN), lambda i: (0, 0, i, 0)),
            pl.BlockSpec((D_IN, D_HID), lambda i: (0, 0)),
            pl.BlockSpec((D_IN, D_HID), lambda i: (0, 0)),
            pl.BlockSpec((1, D_HID), lambda i: (0, 0)),
            pl.BlockSpec((1, D_HID), lambda i: (0, 0)),
        ],
        out_specs=[
            pl.BlockSpec((BN, D_IN), lambda i: (i, 0)),
            pl.BlockSpec((BN, D_IN), lambda i: (i, 0)),
        ],
        out_shape=[
            jax.ShapeDtypeStruct((N_PAD, D_IN), jnp.float32),
            jax.ShapeDtypeStruct((N_PAD, D_IN), jnp.float32),
        ],
    )(agg0, deg, w00, w01, b00, b01)


def _tc2(agg1, deg, w10, w11, b10, b11):
    grid = (N_PAD // BN,)
    return pl.pallas_call(
        _tc2_body,
        grid=grid,
        in_specs=[
            pl.BlockSpec((2, 2, NC, BN, D_IN), lambda i: (0, 0, 0, i, 0)),
            pl.BlockSpec((2, NC, BN, D_IN), lambda i: (0, 0, i, 0)),
            pl.BlockSpec((D_HID, D_OUT), lambda i: (0, 0)),
            pl.BlockSpec((D_HID, D_OUT), lambda i: (0, 0)),
            pl.BlockSpec((1, D_OUT), lambda i: (0, 0)),
            pl.BlockSpec((1, D_OUT), lambda i: (0, 0)),
        ],
        out_specs=pl.BlockSpec((BN, D_OUT), lambda i: (i, 0)),
        out_shape=jax.ShapeDtypeStruct((N_PAD, D_OUT), jnp.float32),
    )(agg1, deg, w10, w11, b10, b11)


def kernel(x, edge_index_r0, edge_index_r1, W0_r0, b0_r0, W0_r1, b0_r1,
           W1_r0, b1_r0, W1_r1, b1_r1):
    E = edge_index_r0.shape[1]
    gp_tile = -(-E // (NW * G * 8)) * 8
    e_pad = NW * G * gp_tile
    ng0 = max(8, int(round(2 * gp_tile * 0.5 / 8)) * 8)
    ng1 = 2 * gp_tile - ng0

    xp = jnp.zeros((N_PAD, D_IN), jnp.float32).at[:N_NODES].set(x)

    def prep(ei):
        idx = ei.astype(jnp.int32)
        pad = jnp.full((e_pad - E,), DUMMY, jnp.int32)
        s = jnp.concatenate([idx[0], pad]).reshape(e_pad // G, G)
        d = jnp.concatenate([idx[1], pad]).reshape(e_pad // G, G)
        return s, d

    s0, d0 = prep(edge_index_r0)
    s1, d1 = prep(edge_index_r1)

    agg0, deg = _make_sc_agg0(ng0, ng1)(xp, s0, d0, s1, d1)
    h0, h1 = _tc1(agg0, deg, W0_r0, W0_r1,
                  b0_r0.reshape(1, -1), b0_r1.reshape(1, -1))
    agg1 = _make_sc_agg1(ng0, ng1)(h0, h1, s0, d0, s1, d1)
    out = _tc2(agg1, deg, W1_r0, W1_r1,
               b1_r0.reshape(1, -1), b1_r1.reshape(1, -1))
    return out[:N_NODES]


# speedup vs baseline: 1.0594x; 1.0594x over previous
import functools

import jax
import jax.numpy as jnp
from jax import lax
from jax.experimental import pallas as pl
from jax.experimental.pallas import tpu as pltpu
from jax.experimental.pallas import tpu_sc as plsc

N_NODES = 10000
D_IN = 128
D_HID = 256
D_OUT = 128

NC = 2
NS = 16
NW = NC * NS
G = 64
N_PAD = 10240
ROWS_PER_TILE = N_PAD // NS
DUMMY = N_NODES
BN = 512
ZR = 16
BF = jnp.bfloat16


def _fill_zero_rows(ref, nrows, dtype):
    if dtype == BF:
        for i in range(nrows):
            for k in range(D_IN // 32):
                ref[i, pl.ds(k * 32, 32)] = jnp.zeros((32,), dtype)
        return

    def body(i, _):
        for k in range(D_IN // 16):
            ref[i, pl.ds(k * 16, 16)] = jnp.zeros((16,), dtype)
        return 0
    lax.fori_loop(0, nrows, body, 0)


def _fill_one_rows(ref, nrows):
    def body(i, _):
        for k in range(D_IN // 16):
            ref[i, pl.ds(k * 16, 16)] = jnp.ones((16,), jnp.float32)
        return 0
    lax.fori_loop(0, nrows, body, 0)


def _zero_slice(zbuf, dst_sh, base, nrows, sem):
    def issue(k, _):
        pltpu.async_copy(zbuf, dst_sh.at[pl.ds(base + k * ZR, ZR)], sem)
        return 0
    lax.fori_loop(0, nrows // ZR, issue, 0)

    def drain(k, _):
        pltpu.make_async_copy(zbuf, dst_sh.at[pl.ds(base, ZR)], sem).wait()
        return 0
    lax.fori_loop(0, nrows // ZR, drain, 0)


def _load_idx(hbm2d, idx_v, row0, ng):
    def body(k, _):
        pltpu.sync_copy(hbm2d.at[pl.ds(row0 + k * 8, 8)], idx_v.at[pl.ds(k * 8, 8)])
        return 0
    lax.fori_loop(0, ng // 8, body, 0)


def _pipelined_agg(tab, src_v, dst_v, rows_v, acc_sh, gs0, gs1, ss0, ss1, ng):
    r0 = rows_v.at[0]
    r1 = rows_v.at[1]
    pltpu.async_copy(tab.at[src_v.at[0]], r0, gs0)
    pltpu.async_copy(tab.at[src_v.at[1]], r1, gs1)

    def pair(p, _):
        g0 = 2 * p
        g1 = g0 + 1
        n0 = jnp.minimum(g0 + 2, ng - 1)
        n1 = jnp.minimum(g1 + 2, ng - 1)
        pltpu.make_async_copy(tab.at[src_v.at[g0]], r0, gs0).wait()
        pltpu.async_copy(r0, acc_sh.at[dst_v.at[g0]], ss0, add=True)
        pltpu.make_async_copy(r0, acc_sh.at[dst_v.at[g0]], ss0).wait()
        pltpu.async_copy(tab.at[src_v.at[n0]], r0, gs0)
        pltpu.make_async_copy(tab.at[src_v.at[g1]], r1, gs1).wait()
        pltpu.async_copy(r1, acc_sh.at[dst_v.at[g1]], ss1, add=True)
        pltpu.make_async_copy(r1, acc_sh.at[dst_v.at[g1]], ss1).wait()
        pltpu.async_copy(tab.at[src_v.at[n1]], r1, gs1)
        return 0

    lax.fori_loop(0, ng // 2, pair, 0)
    pltpu.make_async_copy(tab.at[src_v.at[0]], r0, gs0).wait()
    pltpu.make_async_copy(tab.at[src_v.at[0]], r1, gs1).wait()


def _deg_scatter(ones_v, dst_v, acc_sh, ss0, ng):
    def chunk(cc, _):
        for j in range(8):
            pltpu.async_copy(ones_v, acc_sh.at[dst_v.at[cc * 8 + j]], ss0, add=True)
        for j in range(8):
            pltpu.make_async_copy(ones_v, acc_sh.at[dst_v.at[cc * 8 + j]], ss0).wait()
        return 0

    lax.fori_loop(0, ng // 8, chunk, 0)


def _core_split(c, s, ng0, ng1):
    ng = jnp.where(c == 0, ng0, ng1)
    row0 = jnp.where(c == 0, s * ng0, NS * ng0 + s * ng1)
    return ng, row0


def _mesh():
    return plsc.VectorSubcoreMesh(
        core_axis_name="c", subcore_axis_name="s", num_cores=NC, num_subcores=NS
    )


def _make_sc_agg0(ng0, ng1):
    @functools.partial(
        pl.kernel,
        out_type=jax.ShapeDtypeStruct((2, NC, N_PAD, D_IN), jnp.float32),
        mesh=_mesh(),
        scratch_types=(
            pltpu.VMEM_SHARED((N_PAD, D_IN), jnp.float32),
            pltpu.VMEM((max(ng0, ng1), G), jnp.int32),
            pltpu.VMEM((max(ng0, ng1), G), jnp.int32),
            pltpu.VMEM((2, G, D_IN), jnp.float32),
            pltpu.VMEM((ZR, D_IN), jnp.float32),
            pltpu.SemaphoreType.DMA,
            pltpu.SemaphoreType.DMA,
            pltpu.SemaphoreType.DMA,
            pltpu.SemaphoreType.DMA,
        ),
    )
    def sc_agg0(x_hbm, s0_hbm, d0_hbm, s1_hbm, d1_hbm, agg_out,
                acc_sh, src_v, dst_v, rows_v, zbuf, gs0, gs1, ss0, ss1):
        c = lax.axis_index("c")
        s = lax.axis_index("s")
        ng, row0 = _core_split(c, s, ng0, ng1)
        base = s * ROWS_PER_TILE

        _fill_zero_rows(zbuf, ZR, jnp.float32)
        _zero_slice(zbuf, acc_sh, base, ROWS_PER_TILE, ss0)
        plsc.subcore_barrier()

        for r, (sh, dh) in enumerate(((s0_hbm, d0_hbm), (s1_hbm, d1_hbm))):
            _load_idx(sh, src_v, row0, ng)
            _load_idx(dh, dst_v, row0, ng)
            _pipelined_agg(x_hbm, src_v, dst_v, rows_v, acc_sh,
                           gs0, gs1, ss0, ss1, ng)
            plsc.subcore_barrier()
            pltpu.sync_copy(acc_sh.at[pl.ds(base, ROWS_PER_TILE)],
                            agg_out.at[r, c, pl.ds(base, ROWS_PER_TILE)])
            if r == 0:
                _zero_slice(zbuf, acc_sh, base, ROWS_PER_TILE, ss0)
                plsc.subcore_barrier()

    return sc_agg0


def _make_sc_deg(ng0, ng1):
    @functools.partial(
        pl.kernel,
        out_type=jax.ShapeDtypeStruct((2, NC, N_PAD, D_IN), jnp.float32),
        mesh=_mesh(),
        scratch_types=(
            pltpu.VMEM_SHARED((N_PAD, D_IN), jnp.float32),
            pltpu.VMEM((max(ng0, ng1), G), jnp.int32),
            pltpu.VMEM((G, D_IN), jnp.float32),
            pltpu.VMEM((ZR, D_IN), jnp.float32),
            pltpu.SemaphoreType.DMA,
        ),
    )
    def sc_deg(d0_hbm, d1_hbm, deg_out, acc_sh, dst_v, ones_v, zbuf, ss0):
        c = lax.axis_index("c")
        s = lax.axis_index("s")
        ng, row0 = _core_split(c, s, ng0, ng1)
        base = s * ROWS_PER_TILE

        _fill_zero_rows(zbuf, ZR, jnp.float32)
        _fill_one_rows(ones_v, G)
        _zero_slice(zbuf, acc_sh, base, ROWS_PER_TILE, ss0)
        plsc.subcore_barrier()

        for r, dh in enumerate((d0_hbm, d1_hbm)):
            _load_idx(dh, dst_v, row0, ng)
            _deg_scatter(ones_v, dst_v, acc_sh, ss0, ng)
            plsc.subcore_barrier()
            pltpu.sync_copy(acc_sh.at[pl.ds(base, ROWS_PER_TILE)],
                            deg_out.at[r, c, pl.ds(base, ROWS_PER_TILE)])
            if r == 0:
                _zero_slice(zbuf, acc_sh, base, ROWS_PER_TILE, ss0)
                plsc.subcore_barrier()

    return sc_deg


def _make_sc_agg1(ng0, ng1):
    @functools.partial(
        pl.kernel,
        out_type=jax.ShapeDtypeStruct((2, 2, NC, N_PAD, D_IN), jnp.float32),
        mesh=_mesh(),
        scratch_types=(
            pltpu.VMEM_SHARED((N_PAD, D_IN), jnp.float32),
            pltpu.VMEM((max(ng0, ng1), G), jnp.int32),
            pltpu.VMEM((max(ng0, ng1), G), jnp.int32),
            pltpu.VMEM((2, G, D_IN), jnp.float32),
            pltpu.VMEM((ZR, D_IN), jnp.float32),
            pltpu.SemaphoreType.DMA,
            pltpu.SemaphoreType.DMA,
            pltpu.SemaphoreType.DMA,
            pltpu.SemaphoreType.DMA,
        ),
    )
    def sc_agg1(h0_hbm, h1_hbm, s0_hbm, d0_hbm, s1_hbm, d1_hbm, agg_out,
                acc_sh, src_v, dst_v, rows_v, zbuf, gs0, gs1, ss0, ss1):
        c = lax.axis_index("c")
        s = lax.axis_index("s")
        ng, row0 = _core_split(c, s, ng0, ng1)
        base = s * ROWS_PER_TILE

        _fill_zero_rows(zbuf, ZR, jnp.float32)
        _zero_slice(zbuf, acc_sh, base, ROWS_PER_TILE, ss0)
        plsc.subcore_barrier()

        phases = (
            (0, 0, h0_hbm, s0_hbm, d0_hbm),
            (0, 1, h1_hbm, s0_hbm, d0_hbm),
            (1, 0, h0_hbm, s1_hbm, d1_hbm),
            (1, 1, h1_hbm, s1_hbm, d1_hbm),
        )
        for i, (r, half, tab, sh, dh) in enumerate(phases):
            _load_idx(sh, src_v, row0, ng)
            _load_idx(dh, dst_v, row0, ng)
            _pipelined_agg(tab, src_v, dst_v, rows_v, acc_sh,
                           gs0, gs1, ss0, ss1, ng)
            plsc.subcore_barrier()
            pltpu.sync_copy(acc_sh.at[pl.ds(base, ROWS_PER_TILE)],
                            agg_out.at[r, half, c, pl.ds(base, ROWS_PER_TILE)])
            if i < 3:
                _zero_slice(zbuf, acc_sh, base, ROWS_PER_TILE, ss0)
                plsc.subcore_barrier()

    return sc_agg1


def _norms(deg):
    n0 = 1.0 / jnp.clip(deg[0, 0, :, 0] + deg[0, 1, :, 0], 1.0, None)
    n1 = 1.0 / jnp.clip(deg[1, 0, :, 0] + deg[1, 1, :, 0], 1.0, None)
    return n0, n1


def _tc1_body(agg_ref, deg_ref, w00_ref, w01_ref, b00_ref, b01_ref,
              h0_ref, h1_ref):
    agg = agg_ref[...]
    n0, n1 = _norms(deg_ref[...])
    a0 = (agg[0, 0] + agg[0, 1]) * n0[:, None]
    a1 = (agg[1, 0] + agg[1, 1]) * n1[:, None]
    h = jnp.dot(a0, w00_ref[...], preferred_element_type=jnp.float32)
    h += jnp.dot(a1, w01_ref[...], preferred_element_type=jnp.float32)
    h += b00_ref[...] + b01_ref[...]
    h = jnp.maximum(h, 0.0)
    h0_ref[...] = h[:, :D_IN]
    h1_ref[...] = h[:, D_IN:]


def _tc2_body(agg_ref, deg_ref, w10_ref, w11_ref, b10_ref, b11_ref, out_ref):
    agg = agg_ref[...]
    n0, n1 = _norms(deg_ref[...])
    w10 = w10_ref[...]
    w11 = w11_ref[...]
    acc = jnp.dot((agg[0, 0, 0] + agg[0, 0, 1]) * n0[:, None], w10[:D_IN],
                  preferred_element_type=jnp.float32)
    acc += jnp.dot((agg[0, 1, 0] + agg[0, 1, 1]) * n0[:, None], w10[D_IN:],
                   preferred_element_type=jnp.float32)
    acc += jnp.dot((agg[1, 0, 0] + agg[1, 0, 1]) * n1[:, None], w11[:D_IN],
                   preferred_element_type=jnp.float32)
    acc += jnp.dot((agg[1, 1, 0] + agg[1, 1, 1]) * n1[:, None], w11[D_IN:],
                   preferred_element_type=jnp.float32)
    out_ref[...] = acc + b10_ref[...] + b11_ref[...]


def _tc1(agg0, deg, w00, w01, b00, b01):
    grid = (N_PAD // BN,)
    return pl.pallas_call(
        _tc1_body,
        grid=grid,
        in_specs=[
            pl.BlockSpec((2, NC, BN, D_IN), lambda i: (0, 0, i, 0)),
            pl.BlockSpec((2, NC, BN, D_IN), lambda i: (0, 0, i, 0)),
            pl.BlockSpec((D_IN, D_HID), lambda i: (0, 0)),
            pl.BlockSpec((D_IN, D_HID), lambda i: (0, 0)),
            pl.BlockSpec((1, D_HID), lambda i: (0, 0)),
            pl.BlockSpec((1, D_HID), lambda i: (0, 0)),
        ],
        out_specs=[
            pl.BlockSpec((BN, D_IN), lambda i: (i, 0)),
            pl.BlockSpec((BN, D_IN), lambda i: (i, 0)),
        ],
        out_shape=[
            jax.ShapeDtypeStruct((N_PAD, D_IN), jnp.float32),
            jax.ShapeDtypeStruct((N_PAD, D_IN), jnp.float32),
        ],
    )(agg0, deg, w00, w01, b00, b01)


def _tc2(agg1, deg, w10, w11, b10, b11):
    grid = (N_PAD // BN,)
    return pl.pallas_call(
        _tc2_body,
        grid=grid,
        in_specs=[
            pl.BlockSpec((2, 2, NC, BN, D_IN), lambda i: (0, 0, 0, i, 0)),
            pl.BlockSpec((2, NC, BN, D_IN), lambda i: (0, 0, i, 0)),
            pl.BlockSpec((D_HID, D_OUT), lambda i: (0, 0)),
            pl.BlockSpec((D_HID, D_OUT), lambda i: (0, 0)),
            pl.BlockSpec((1, D_OUT), lambda i: (0, 0)),
            pl.BlockSpec((1, D_OUT), lambda i: (0, 0)),
        ],
        out_specs=pl.BlockSpec((BN, D_OUT), lambda i: (i, 0)),
        out_shape=jax.ShapeDtypeStruct((N_PAD, D_OUT), jnp.float32),
    )(agg1, deg, w10, w11, b10, b11)


def kernel(x, edge_index_r0, edge_index_r1, W0_r0, b0_r0, W0_r1, b0_r1,
           W1_r0, b1_r0, W1_r1, b1_r1):
    E = edge_index_r0.shape[1]
    gp_tile = -(-E // (NW * G * 8)) * 8
    e_pad = NW * G * gp_tile
    ng0 = max(8, int(round(2 * gp_tile * 0.7 / 8)) * 8)
    ng1 = 2 * gp_tile - ng0

    xp = jnp.zeros((N_PAD, D_IN), jnp.float32).at[:N_NODES].set(x)

    def prep(ei):
        idx = ei.astype(jnp.int32)
        pad = jnp.full((e_pad - E,), DUMMY, jnp.int32)
        s = jnp.concatenate([idx[0], pad]).reshape(e_pad // G, G)
        d = jnp.concatenate([idx[1], pad]).reshape(e_pad // G, G)
        return s, d

    s0, d0 = prep(edge_index_r0)
    s1, d1 = prep(edge_index_r1)

    agg0 = _make_sc_agg0(ng0, ng1)(xp, s0, d0, s1, d1)
    deg = _make_sc_deg(ng0, ng1)(d0, d1)
    h0, h1 = _tc1(agg0, deg, W0_r0, W0_r1,
                  b0_r0.reshape(1, -1), b0_r1.reshape(1, -1))
    agg1 = _make_sc_agg1(ng0, ng1)(h0, h1, s0, d0, s1, d1)
    out = _tc2(agg1, deg, W1_r0, W1_r1,
               b1_r0.reshape(1, -1), b1_r1.reshape(1, -1))
    return out[:N_NODES]


# speedup vs baseline: 1.0648x; 1.0051x over previous
import functools

import jax
import jax.numpy as jnp
from jax import lax
from jax.experimental import pallas as pl
from jax.experimental.pallas import tpu as pltpu
from jax.experimental.pallas import tpu_sc as plsc

N_NODES = 10000
D_IN = 128
D_HID = 256
D_OUT = 128

NC = 2
NS = 16
NW = NC * NS
G = 64
N_PAD = 10240
ROWS_PER_TILE = N_PAD // NS
DUMMY = N_NODES
BN = 512
ZR = 16
BF = jnp.bfloat16


def _fill_zero_rows(ref, nrows, dtype):
    if dtype == BF:
        for i in range(nrows):
            for k in range(D_IN // 32):
                ref[i, pl.ds(k * 32, 32)] = jnp.zeros((32,), dtype)
        return

    def body(i, _):
        for k in range(D_IN // 16):
            ref[i, pl.ds(k * 16, 16)] = jnp.zeros((16,), dtype)
        return 0
    lax.fori_loop(0, nrows, body, 0)


def _fill_one_rows(ref, nrows):
    def body(i, _):
        for k in range(D_IN // 16):
            ref[i, pl.ds(k * 16, 16)] = jnp.ones((16,), jnp.float32)
        return 0
    lax.fori_loop(0, nrows, body, 0)


def _zero_slice(zbuf, dst_sh, base, nrows, sem):
    def issue(k, _):
        pltpu.async_copy(zbuf, dst_sh.at[pl.ds(base + k * ZR, ZR)], sem)
        return 0
    lax.fori_loop(0, nrows // ZR, issue, 0)

    def drain(k, _):
        pltpu.make_async_copy(zbuf, dst_sh.at[pl.ds(base, ZR)], sem).wait()
        return 0
    lax.fori_loop(0, nrows // ZR, drain, 0)


def _load_idx(hbm2d, idx_v, row0, ngt):
    def body(k, _):
        pltpu.sync_copy(hbm2d.at[pl.ds(row0 + k * 8, 8)], idx_v.at[pl.ds(k * 8, 8)])
        return 0
    lax.fori_loop(0, ngt // 8, body, 0)


def _pipelined_agg(tab, src_v, dst_v, rows_v, acc_sh, sems, ngt):
    r = [rows_v.at[j] for j in range(3)]

    def gather(g, j):
        pltpu.async_copy(tab.at[src_v.at[g]], r[j], sems[j])

    def wait_g(j):
        pltpu.make_async_copy(tab.at[src_v.at[0]], r[j], sems[j]).wait()

    def scatter(g, j):
        pltpu.async_copy(r[j], acc_sh.at[dst_v.at[g]], sems[j], add=True)

    def wait_s(j):
        pltpu.make_async_copy(r[j], acc_sh.at[dst_v.at[0]], sems[j]).wait()

    gather(0, 0)
    gather(1, 1)
    gather(2, 2)
    wait_g(0)
    scatter(0, 0)

    def block(p, _):
        gbase = 1 + 3 * p
        for j3 in range(3):
            g = gbase + j3
            wait_s(j3)
            gather(jnp.minimum(g + 2, ngt - 1), j3)
            wait_g((1 + j3) % 3)
            scatter(g, (1 + j3) % 3)
        return 0

    lax.fori_loop(0, (ngt - 2) // 3, block, 0)

    wait_s(0)
    wait_g(1)
    scatter(ngt - 1, 1)
    wait_g(2)
    wait_s(1)


def _deg_scatter(ones_v, dst_v, acc_sh, ss0, ngt):
    def chunk(cc, _):
        for j in range(8):
            pltpu.async_copy(ones_v, acc_sh.at[dst_v.at[cc * 8 + j]], ss0, add=True)
        for j in range(8):
            pltpu.make_async_copy(ones_v, acc_sh.at[dst_v.at[cc * 8 + j]], ss0).wait()
        return 0

    lax.fori_loop(0, ngt // 8, chunk, 0)


def _mesh():
    return plsc.VectorSubcoreMesh(
        core_axis_name="c", subcore_axis_name="s", num_cores=NC, num_subcores=NS
    )


def _make_sc_agg0(ng):
    @functools.partial(
        pl.kernel,
        out_type=jax.ShapeDtypeStruct((2, NC, N_PAD, D_IN), jnp.float32),
        mesh=_mesh(),
        scratch_types=(
            pltpu.VMEM_SHARED((N_PAD, D_IN), jnp.float32),
            pltpu.VMEM((ng, G), jnp.int32),
            pltpu.VMEM((ng, G), jnp.int32),
            pltpu.VMEM((3, G, D_IN), jnp.float32),
            pltpu.VMEM((ZR, D_IN), jnp.float32),
        ) + tuple([pltpu.SemaphoreType.DMA] * 3) + (
        ),
    )
    def sc_agg0(x_hbm, s0_hbm, d0_hbm, s1_hbm, d1_hbm, agg_out,
                acc_sh, src_v, dst_v, rows_v, zbuf, m0, m1, m2):
        c = lax.axis_index("c")
        s = lax.axis_index("s")
        ngt = jnp.where(c < NC, ng, 0)
        row0 = (c * NS + s) * ngt
        base = s * ROWS_PER_TILE

        _fill_zero_rows(zbuf, ZR, jnp.float32)
        _zero_slice(zbuf, acc_sh, base, ROWS_PER_TILE, m0)
        plsc.subcore_barrier()

        for r, (sh, dh) in enumerate(((s0_hbm, d0_hbm), (s1_hbm, d1_hbm))):
            _load_idx(sh, src_v, row0, ngt)
            _load_idx(dh, dst_v, row0, ngt)
            _pipelined_agg(x_hbm, src_v, dst_v, rows_v, acc_sh,
                           (m0, m1, m2), ngt)
            plsc.subcore_barrier()
            pltpu.sync_copy(acc_sh.at[pl.ds(base, ROWS_PER_TILE)],
                            agg_out.at[r, c, pl.ds(base, ROWS_PER_TILE)])
            if r == 0:
                _zero_slice(zbuf, acc_sh, base, ROWS_PER_TILE, m0)
                plsc.subcore_barrier()

    return sc_agg0


def _make_sc_deg(ng):
    @functools.partial(
        pl.kernel,
        out_type=jax.ShapeDtypeStruct((2, NC, N_PAD, D_IN), jnp.float32),
        mesh=_mesh(),
        scratch_types=(
            pltpu.VMEM_SHARED((N_PAD, D_IN), jnp.float32),
            pltpu.VMEM((ng, G), jnp.int32),
            pltpu.VMEM((G, D_IN), jnp.float32),
            pltpu.VMEM((ZR, D_IN), jnp.float32),
            pltpu.SemaphoreType.DMA,
        ),
    )
    def sc_deg(d0_hbm, d1_hbm, deg_out, acc_sh, dst_v, ones_v, zbuf, ss0):
        c = lax.axis_index("c")
        s = lax.axis_index("s")
        ngt = jnp.where(c < NC, ng, 0)
        row0 = (c * NS + s) * ngt
        base = s * ROWS_PER_TILE

        _fill_zero_rows(zbuf, ZR, jnp.float32)
        _fill_one_rows(ones_v, G)
        _zero_slice(zbuf, acc_sh, base, ROWS_PER_TILE, ss0)
        plsc.subcore_barrier()

        for r, dh in enumerate((d0_hbm, d1_hbm)):
            _load_idx(dh, dst_v, row0, ngt)
            _deg_scatter(ones_v, dst_v, acc_sh, ss0, ngt)
            plsc.subcore_barrier()
            pltpu.sync_copy(acc_sh.at[pl.ds(base, ROWS_PER_TILE)],
                            deg_out.at[r, c, pl.ds(base, ROWS_PER_TILE)])
            if r == 0:
                _zero_slice(zbuf, acc_sh, base, ROWS_PER_TILE, ss0)
                plsc.subcore_barrier()

    return sc_deg


def _make_sc_agg1(ng):
    @functools.partial(
        pl.kernel,
        out_type=jax.ShapeDtypeStruct((2, 2, NC, N_PAD, D_IN), jnp.float32),
        mesh=_mesh(),
        scratch_types=(
            pltpu.VMEM_SHARED((N_PAD, D_IN), jnp.float32),
            pltpu.VMEM((ng, G), jnp.int32),
            pltpu.VMEM((ng, G), jnp.int32),
            pltpu.VMEM((3, G, D_IN), jnp.float32),
            pltpu.VMEM((ZR, D_IN), jnp.float32),
        ) + tuple([pltpu.SemaphoreType.DMA] * 3) + (
        ),
    )
    def sc_agg1(h0_hbm, h1_hbm, s0_hbm, d0_hbm, s1_hbm, d1_hbm, agg_out,
                acc_sh, src_v, dst_v, rows_v, zbuf, m0, m1, m2):
        c = lax.axis_index("c")
        s = lax.axis_index("s")
        ngt = jnp.where(c < NC, ng, 0)
        row0 = (c * NS + s) * ngt
        base = s * ROWS_PER_TILE

        _fill_zero_rows(zbuf, ZR, jnp.float32)
        _zero_slice(zbuf, acc_sh, base, ROWS_PER_TILE, m0)
        plsc.subcore_barrier()

        phases = (
            (0, 0, h0_hbm, s0_hbm, d0_hbm),
            (0, 1, h1_hbm, s0_hbm, d0_hbm),
            (1, 0, h0_hbm, s1_hbm, d1_hbm),
            (1, 1, h1_hbm, s1_hbm, d1_hbm),
        )
        for i, (r, half, tab, sh, dh) in enumerate(phases):
            _load_idx(sh, src_v, row0, ngt)
            _load_idx(dh, dst_v, row0, ngt)
            _pipelined_agg(tab, src_v, dst_v, rows_v, acc_sh,
                           (m0, m1, m2), ngt)
            plsc.subcore_barrier()
            pltpu.sync_copy(acc_sh.at[pl.ds(base, ROWS_PER_TILE)],
                            agg_out.at[r, half, c, pl.ds(base, ROWS_PER_TILE)])
            if i < 3:
                _zero_slice(zbuf, acc_sh, base, ROWS_PER_TILE, m0)
                plsc.subcore_barrier()

    return sc_agg1


def _norms(deg):
    n0 = 1.0 / jnp.clip(deg[0, 0, :, 0] + deg[0, 1, :, 0], 1.0, None)
    n1 = 1.0 / jnp.clip(deg[1, 0, :, 0] + deg[1, 1, :, 0], 1.0, None)
    return n0, n1


def _tc1_body(agg_ref, deg_ref, w00_ref, w01_ref, b00_ref, b01_ref,
              h0_ref, h1_ref):
    agg = agg_ref[...]
    n0, n1 = _norms(deg_ref[...])
    a0 = (agg[0, 0] + agg[0, 1]) * n0[:, None]
    a1 = (agg[1, 0] + agg[1, 1]) * n1[:, None]
    h = jnp.dot(a0, w00_ref[...], preferred_element_type=jnp.float32)
    h += jnp.dot(a1, w01_ref[...], preferred_element_type=jnp.float32)
    h += b00_ref[...] + b01_ref[...]
    h = jnp.maximum(h, 0.0)
    h0_ref[...] = h[:, :D_IN]
    h1_ref[...] = h[:, D_IN:]


def _tc2_body(agg_ref, deg_ref, w10_ref, w11_ref, b10_ref, b11_ref, out_ref):
    agg = agg_ref[...]
    n0, n1 = _norms(deg_ref[...])
    w10 = w10_ref[...]
    w11 = w11_ref[...]
    acc = jnp.dot((agg[0, 0, 0] + agg[0, 0, 1]) * n0[:, None], w10[:D_IN],
                  preferred_element_type=jnp.float32)
    acc += jnp.dot((agg[0, 1, 0] + agg[0, 1, 1]) * n0[:, None], w10[D_IN:],
                   preferred_element_type=jnp.float32)
    acc += jnp.dot((agg[1, 0, 0] + agg[1, 0, 1]) * n1[:, None], w11[:D_IN],
                   preferred_element_type=jnp.float32)
    acc += jnp.dot((agg[1, 1, 0] + agg[1, 1, 1]) * n1[:, None], w11[D_IN:],
                   preferred_element_type=jnp.float32)
    out_ref[...] = acc + b10_ref[...] + b11_ref[...]


def _tc1(agg0, deg, w00, w01, b00, b01):
    grid = (N_PAD // BN,)
    return pl.pallas_call(
        _tc1_body,
        grid=grid,
        in_specs=[
            pl.BlockSpec((2, NC, BN, D_IN), lambda i: (0, 0, i, 0)),
            pl.BlockSpec((2, NC, BN, D_IN), lambda i: (0, 0, i, 0)),
            pl.BlockSpec((D_IN, D_HID), lambda i: (0, 0)),
            pl.BlockSpec((D_IN, D_HID), lambda i: (0, 0)),
            pl.BlockSpec((1, D_HID), lambda i: (0, 0)),
            pl.BlockSpec((1, D_HID), lambda i: (0, 0)),
        ],
        out_specs=[
            pl.BlockSpec((BN, D_IN), lambda i: (i, 0)),
            pl.BlockSpec((BN, D_IN), lambda i: (i, 0)),
        ],
        out_shape=[
            jax.ShapeDtypeStruct((N_PAD, D_IN), jnp.float32),
            jax.ShapeDtypeStruct((N_PAD, D_IN), jnp.float32),
        ],
    )(agg0, deg, w00, w01, b00, b01)


def _tc2(agg1, deg, w10, w11, b10, b11):
    grid = (N_PAD // BN,)
    return pl.pallas_call(
        _tc2_body,
        grid=grid,
        in_specs=[
            pl.BlockSpec((2, 2, NC, BN, D_IN), lambda i: (0, 0, 0, i, 0)),
            pl.BlockSpec((2, NC, BN, D_IN), lambda i: (0, 0, i, 0)),
            pl.BlockSpec((D_HID, D_OUT), lambda i: (0, 0)),
            pl.BlockSpec((D_HID, D_OUT), lambda i: (0, 0)),
            pl.BlockSpec((1, D_OUT), lambda i: (0, 0)),
            pl.BlockSpec((1, D_OUT), lambda i: (0, 0)),
        ],
        out_specs=pl.BlockSpec((BN, D_OUT), lambda i: (i, 0)),
        out_shape=jax.ShapeDtypeStruct((N_PAD, D_OUT), jnp.float32),
    )(agg1, deg, w10, w11, b10, b11)


def kernel(x, edge_index_r0, edge_index_r1, W0_r0, b0_r0, W0_r1, b0_r1,
           W1_r0, b1_r0, W1_r1, b1_r1):
    E = edge_index_r0.shape[1]
    gp_tile = -(-E // (NW * G * 8)) * 8
    while gp_tile % 3 != 2:
        gp_tile += 8
    e_pad = NW * G * gp_tile


    xp = jnp.zeros((N_PAD, D_IN), jnp.float32).at[:N_NODES].set(x)

    def prep(ei):
        idx = ei.astype(jnp.int32)
        pad = jnp.full((e_pad - E,), DUMMY, jnp.int32)
        s = jnp.concatenate([idx[0], pad]).reshape(e_pad // G, G)
        d = jnp.concatenate([idx[1], pad]).reshape(e_pad // G, G)
        return s, d

    s0, d0 = prep(edge_index_r0)
    s1, d1 = prep(edge_index_r1)

    agg0 = _make_sc_agg0(gp_tile)(xp, s0, d0, s1, d1)
    deg = _make_sc_deg(gp_tile)(d0, d1)
    h0, h1 = _tc1(agg0, deg, W0_r0, W0_r1,
                  b0_r0.reshape(1, -1), b0_r1.reshape(1, -1))
    agg1 = _make_sc_agg1(gp_tile)(h0, h1, s0, d0, s1, d1)
    out = _tc2(agg1, deg, W1_r0, W1_r1,
               b1_r0.reshape(1, -1), b1_r1.reshape(1, -1))
    return out[:N_NODES]


# speedup vs baseline: 1.4771x; 1.3872x over previous
import functools

import jax
import jax.numpy as jnp
from jax import lax
from jax.experimental import pallas as pl
from jax.experimental.pallas import tpu as pltpu
from jax.experimental.pallas import tpu_sc as plsc

N_NODES = 10000
D_IN = 128
D_HID = 256
D_OUT = 128

NC = 2
NS = 16
NW = NC * NS
G = 64
N_PAD = 10240
ROWS_PER_TILE = N_PAD // NS
DUMMY = N_NODES
BN = 512
ZR = 16
BF = jnp.bfloat16


def _fill_zero_rows(ref, nrows, dtype):
    if dtype == BF:
        for i in range(nrows):
            for k in range(D_IN // 32):
                ref[i, pl.ds(k * 32, 32)] = jnp.zeros((32,), dtype)
        return

    def body(i, _):
        for k in range(D_IN // 16):
            ref[i, pl.ds(k * 16, 16)] = jnp.zeros((16,), dtype)
        return 0
    lax.fori_loop(0, nrows, body, 0)


def _fill_one_rows(ref, nrows):
    def body(i, _):
        for k in range(D_IN // 16):
            ref[i, pl.ds(k * 16, 16)] = jnp.ones((16,), jnp.float32)
        return 0
    lax.fori_loop(0, nrows, body, 0)


def _zero_slice(zbuf, dst_sh, base, nrows, sem):
    def issue(k, _):
        pltpu.async_copy(zbuf, dst_sh.at[pl.ds(base + k * ZR, ZR)], sem)
        return 0
    lax.fori_loop(0, nrows // ZR, issue, 0)

    def drain(k, _):
        pltpu.make_async_copy(zbuf, dst_sh.at[pl.ds(base, ZR)], sem).wait()
        return 0
    lax.fori_loop(0, nrows // ZR, drain, 0)


def _load_idx(hbm2d, idx_v, row0, ngt):
    def body(k, _):
        pltpu.sync_copy(hbm2d.at[pl.ds(row0 + k * 8, 8)], idx_v.at[pl.ds(k * 8, 8)])
        return 0
    lax.fori_loop(0, ngt // 8, body, 0)


def _pipelined_agg(tab, src_v, dst_v, rows_v, acc_sh, sems, ngt):
    r = [rows_v.at[j] for j in range(3)]

    def gather(g, j):
        pltpu.async_copy(tab.at[src_v.at[g]], r[j], sems[j])

    def wait_g(j):
        pltpu.make_async_copy(tab.at[src_v.at[0]], r[j], sems[j]).wait()

    def scatter(g, j):
        pltpu.async_copy(r[j], acc_sh.at[dst_v.at[g]], sems[j], add=True)

    def wait_s(j):
        pltpu.make_async_copy(r[j], acc_sh.at[dst_v.at[0]], sems[j]).wait()

    gather(0, 0)
    gather(1, 1)
    gather(2, 2)
    wait_g(0)
    scatter(0, 0)

    def block(p, _):
        gbase = 1 + 3 * p
        for j3 in range(3):
            g = gbase + j3
            wait_s(j3)
            gather(jnp.minimum(g + 2, ngt - 1), j3)
            wait_g((1 + j3) % 3)
            scatter(g, (1 + j3) % 3)
        return 0

    lax.fori_loop(0, (ngt - 2) // 3, block, 0)

    wait_s(0)
    wait_g(1)
    scatter(ngt - 1, 1)
    wait_g(2)
    wait_s(1)


def _deg_scatter(ones_v, dst_v, acc_sh, ss0, ngt):
    def chunk(cc, _):
        for j in range(8):
            pltpu.async_copy(ones_v, acc_sh.at[dst_v.at[cc * 8 + j]], ss0, add=True)
        for j in range(8):
            pltpu.make_async_copy(ones_v, acc_sh.at[dst_v.at[cc * 8 + j]], ss0).wait()
        return 0

    lax.fori_loop(0, ngt // 8, chunk, 0)


def _mesh():
    return plsc.VectorSubcoreMesh(
        core_axis_name="c", subcore_axis_name="s", num_cores=NC, num_subcores=NS
    )


def _make_sc_agg(ng):
    @functools.partial(
        pl.kernel,
        out_type=jax.ShapeDtypeStruct((2, NC, N_PAD, D_IN), jnp.float32),
        mesh=_mesh(),
        scratch_types=(
            pltpu.VMEM_SHARED((N_PAD, D_IN), jnp.float32),
            pltpu.VMEM((ng, G), jnp.int32),
            pltpu.VMEM((ng, G), jnp.int32),
            pltpu.VMEM((3, G, D_IN), jnp.float32),
            pltpu.VMEM((ZR, D_IN), jnp.float32),
        ) + tuple([pltpu.SemaphoreType.DMA] * 3) + (
        ),
    )
    def sc_agg(t0_hbm, t1_hbm, s0_hbm, d0_hbm, s1_hbm, d1_hbm, agg_out,
               acc_sh, src_v, dst_v, rows_v, zbuf, m0, m1, m2):
        c = lax.axis_index("c")
        s = lax.axis_index("s")
        ngt = jnp.where(c < NC, ng, 0)
        row0 = (c * NS + s) * ngt
        base = s * ROWS_PER_TILE

        _fill_zero_rows(zbuf, ZR, jnp.float32)
        _zero_slice(zbuf, acc_sh, base, ROWS_PER_TILE, m0)
        plsc.subcore_barrier()

        for r, (tab, sh, dh) in enumerate(((t0_hbm, s0_hbm, d0_hbm),
                                           (t1_hbm, s1_hbm, d1_hbm))):
            _load_idx(sh, src_v, row0, ngt)
            _load_idx(dh, dst_v, row0, ngt)
            _pipelined_agg(tab, src_v, dst_v, rows_v, acc_sh,
                           (m0, m1, m2), ngt)
            plsc.subcore_barrier()
            pltpu.sync_copy(acc_sh.at[pl.ds(base, ROWS_PER_TILE)],
                            agg_out.at[r, c, pl.ds(base, ROWS_PER_TILE)])
            if r == 0:
                _zero_slice(zbuf, acc_sh, base, ROWS_PER_TILE, m0)
                plsc.subcore_barrier()

    return sc_agg


def _make_sc_deg(ng):
    @functools.partial(
        pl.kernel,
        out_type=jax.ShapeDtypeStruct((2, NC, N_PAD, D_IN), jnp.float32),
        mesh=_mesh(),
        scratch_types=(
            pltpu.VMEM_SHARED((N_PAD, D_IN), jnp.float32),
            pltpu.VMEM((ng, G), jnp.int32),
            pltpu.VMEM((G, D_IN), jnp.float32),
            pltpu.VMEM((ZR, D_IN), jnp.float32),
            pltpu.SemaphoreType.DMA,
        ),
    )
    def sc_deg(d0_hbm, d1_hbm, deg_out, acc_sh, dst_v, ones_v, zbuf, ss0):
        c = lax.axis_index("c")
        s = lax.axis_index("s")
        ngt = jnp.where(c < NC, ng, 0)
        row0 = (c * NS + s) * ngt
        base = s * ROWS_PER_TILE

        _fill_zero_rows(zbuf, ZR, jnp.float32)
        _fill_one_rows(ones_v, G)
        _zero_slice(zbuf, acc_sh, base, ROWS_PER_TILE, ss0)
        plsc.subcore_barrier()

        for r, dh in enumerate((d0_hbm, d1_hbm)):
            _load_idx(dh, dst_v, row0, ngt)
            _deg_scatter(ones_v, dst_v, acc_sh, ss0, ngt)
            plsc.subcore_barrier()
            pltpu.sync_copy(acc_sh.at[pl.ds(base, ROWS_PER_TILE)],
                            deg_out.at[r, c, pl.ds(base, ROWS_PER_TILE)])
            if r == 0:
                _zero_slice(zbuf, acc_sh, base, ROWS_PER_TILE, ss0)
                plsc.subcore_barrier()

    return sc_deg


def _norms(deg):
    n0 = 1.0 / jnp.clip(deg[0, 0, :, 0] + deg[0, 1, :, 0], 1.0, None)
    n1 = 1.0 / jnp.clip(deg[1, 0, :, 0] + deg[1, 1, :, 0], 1.0, None)
    return n0, n1


def _tc1_body(agg_ref, deg_ref, w00_ref, w01_ref, b00_ref, b01_ref,
              w10_ref, w11_ref, m0_ref, m1_ref):
    agg = agg_ref[...]
    n0, n1 = _norms(deg_ref[...])
    a0 = (agg[0, 0] + agg[0, 1]) * n0[:, None]
    a1 = (agg[1, 0] + agg[1, 1]) * n1[:, None]
    h = jnp.dot(a0, w00_ref[...], preferred_element_type=jnp.float32)
    h += jnp.dot(a1, w01_ref[...], preferred_element_type=jnp.float32)
    h += b00_ref[...] + b01_ref[...]
    h = jnp.maximum(h, 0.0)
    m0_ref[...] = jnp.dot(h, w10_ref[...], preferred_element_type=jnp.float32)
    m1_ref[...] = jnp.dot(h, w11_ref[...], preferred_element_type=jnp.float32)


def _tc2_body(agg_ref, deg_ref, b10_ref, b11_ref, out_ref):
    agg = agg_ref[...]
    n0, n1 = _norms(deg_ref[...])
    out = (agg[0, 0] + agg[0, 1]) * n0[:, None]
    out += (agg[1, 0] + agg[1, 1]) * n1[:, None]
    out_ref[...] = out + b10_ref[...] + b11_ref[...]


def _tc1(agg0, deg, w00, w01, b00, b01, w10, w11):
    grid = (N_PAD // BN,)
    return pl.pallas_call(
        _tc1_body,
        grid=grid,
        in_specs=[
            pl.BlockSpec((2, NC, BN, D_IN), lambda i: (0, 0, i, 0)),
            pl.BlockSpec((2, NC, BN, D_IN), lambda i: (0, 0, i, 0)),
            pl.BlockSpec((D_IN, D_HID), lambda i: (0, 0)),
            pl.BlockSpec((D_IN, D_HID), lambda i: (0, 0)),
            pl.BlockSpec((1, D_HID), lambda i: (0, 0)),
            pl.BlockSpec((1, D_HID), lambda i: (0, 0)),
            pl.BlockSpec((D_HID, D_OUT), lambda i: (0, 0)),
            pl.BlockSpec((D_HID, D_OUT), lambda i: (0, 0)),
        ],
        out_specs=[
            pl.BlockSpec((BN, D_OUT), lambda i: (i, 0)),
            pl.BlockSpec((BN, D_OUT), lambda i: (i, 0)),
        ],
        out_shape=[
            jax.ShapeDtypeStruct((N_PAD, D_OUT), jnp.float32),
            jax.ShapeDtypeStruct((N_PAD, D_OUT), jnp.float32),
        ],
    )(agg0, deg, w00, w01, b00, b01, w10, w11)


def _tc2(agg1, deg, b10, b11):
    grid = (N_PAD // BN,)
    return pl.pallas_call(
        _tc2_body,
        grid=grid,
        in_specs=[
            pl.BlockSpec((2, NC, BN, D_OUT), lambda i: (0, 0, i, 0)),
            pl.BlockSpec((2, NC, BN, D_IN), lambda i: (0, 0, i, 0)),
            pl.BlockSpec((1, D_OUT), lambda i: (0, 0)),
            pl.BlockSpec((1, D_OUT), lambda i: (0, 0)),
        ],
        out_specs=pl.BlockSpec((BN, D_OUT), lambda i: (i, 0)),
        out_shape=jax.ShapeDtypeStruct((N_PAD, D_OUT), jnp.float32),
    )(agg1, deg, b10, b11)


def kernel(x, edge_index_r0, edge_index_r1, W0_r0, b0_r0, W0_r1, b0_r1,
           W1_r0, b1_r0, W1_r1, b1_r1):
    E = edge_index_r0.shape[1]
    gp_tile = -(-E // (NW * G * 8)) * 8
    while gp_tile % 3 != 2:
        gp_tile += 8
    e_pad = NW * G * gp_tile


    xp = jnp.zeros((N_PAD, D_IN), jnp.float32).at[:N_NODES].set(x)

    def prep(ei):
        idx = ei.astype(jnp.int32)
        pad = jnp.full((e_pad - E,), DUMMY, jnp.int32)
        s = jnp.concatenate([idx[0], pad]).reshape(e_pad // G, G)
        d = jnp.concatenate([idx[1], pad]).reshape(e_pad // G, G)
        return s, d

    s0, d0 = prep(edge_index_r0)
    s1, d1 = prep(edge_index_r1)

    agg0 = _make_sc_agg(gp_tile)(xp, xp, s0, d0, s1, d1)
    deg = _make_sc_deg(gp_tile)(d0, d1)
    m0, m1 = _tc1(agg0, deg, W0_r0, W0_r1,
                  b0_r0.reshape(1, -1), b0_r1.reshape(1, -1), W1_r0, W1_r1)
    agg1 = _make_sc_agg(gp_tile)(m0, m1, s0, d0, s1, d1)
    out = _tc2(agg1, deg, b1_r0.reshape(1, -1), b1_r1.reshape(1, -1))
    return out[:N_NODES]


# speedup vs baseline: 1.4834x; 1.0042x over previous
import functools

import jax
import jax.numpy as jnp
from jax import lax
from jax.experimental import pallas as pl
from jax.experimental.pallas import tpu as pltpu
from jax.experimental.pallas import tpu_sc as plsc

N_NODES = 10000
D_IN = 128
D_HID = 256
D_OUT = 128

NC = 2
NS = 16
NW = NC * NS
G = 64
N_PAD = 10240
ROWS_PER_TILE = N_PAD // NS
DUMMY = N_NODES
BN = 512
ZR = 16
BF = jnp.bfloat16


def _fill_zero_rows(ref, nrows, dtype):
    if dtype == BF:
        for i in range(nrows):
            for k in range(D_IN // 32):
                ref[i, pl.ds(k * 32, 32)] = jnp.zeros((32,), dtype)
        return

    def body(i, _):
        for k in range(D_IN // 16):
            ref[i, pl.ds(k * 16, 16)] = jnp.zeros((16,), dtype)
        return 0
    lax.fori_loop(0, nrows, body, 0)


def _fill_one_rows(ref, nrows):
    def body(i, _):
        for k in range(D_IN // 16):
            ref[i, pl.ds(k * 16, 16)] = jnp.ones((16,), jnp.float32)
        return 0
    lax.fori_loop(0, nrows, body, 0)


def _make_sc_agg0deg(ng):
    @functools.partial(
        pl.kernel,
        out_type=(
            jax.ShapeDtypeStruct((2, NC, N_PAD, D_IN), jnp.float32),
            jax.ShapeDtypeStruct((2, NC, N_PAD, D_IN), jnp.float32),
        ),
        mesh=_mesh(),
        scratch_types=(
            pltpu.VMEM_SHARED((N_PAD, D_IN), jnp.float32),
            pltpu.VMEM((ng, G), jnp.int32),
            pltpu.VMEM((ng, G), jnp.int32),
            pltpu.VMEM((3, G, D_IN), jnp.float32),
            pltpu.VMEM((ZR, D_IN), jnp.float32),
        ) + tuple([pltpu.SemaphoreType.DMA] * 3) + (
        ),
    )
    def sc_agg0deg(x_hbm, s0_hbm, d0_hbm, s1_hbm, d1_hbm, agg_out, deg_out,
                   acc_sh, src_v, dst_v, rows_v, zbuf, m0, m1, m2):
        c = lax.axis_index("c")
        s = lax.axis_index("s")
        ngt = jnp.where(c < NC, ng, 0)
        row0 = (c * NS + s) * ngt
        base = s * ROWS_PER_TILE

        _fill_zero_rows(zbuf, ZR, jnp.float32)
        _zero_slice(zbuf, acc_sh, base, ROWS_PER_TILE, m0)
        plsc.subcore_barrier()

        for r, (sh, dh) in enumerate(((s0_hbm, d0_hbm), (s1_hbm, d1_hbm))):
            _load_idx(sh, src_v, row0, ngt)
            _load_idx(dh, dst_v, row0, ngt)
            _pipelined_agg(x_hbm, src_v, dst_v, rows_v, acc_sh,
                           (m0, m1, m2), ngt)
            plsc.subcore_barrier()
            pltpu.sync_copy(acc_sh.at[pl.ds(base, ROWS_PER_TILE)],
                            agg_out.at[r, c, pl.ds(base, ROWS_PER_TILE)])
            _zero_slice(zbuf, acc_sh, base, ROWS_PER_TILE, m0)
            plsc.subcore_barrier()

        ones_v = rows_v.at[0]
        _fill_one_rows(ones_v, G)
        for r, dh in enumerate((d0_hbm, d1_hbm)):
            _load_idx(dh, dst_v, row0, ngt)
            _deg_scatter(ones_v, dst_v, acc_sh, m1, ngt)
            plsc.subcore_barrier()
            pltpu.sync_copy(acc_sh.at[pl.ds(base, ROWS_PER_TILE)],
                            deg_out.at[r, c, pl.ds(base, ROWS_PER_TILE)])
            if r == 0:
                _zero_slice(zbuf, acc_sh, base, ROWS_PER_TILE, m0)
                plsc.subcore_barrier()

    return sc_agg0deg


def _zero_slice(zbuf, dst_sh, base, nrows, sem):
    def issue(k, _):
        pltpu.async_copy(zbuf, dst_sh.at[pl.ds(base + k * ZR, ZR)], sem)
        return 0
    lax.fori_loop(0, nrows // ZR, issue, 0)

    def drain(k, _):
        pltpu.make_async_copy(zbuf, dst_sh.at[pl.ds(base, ZR)], sem).wait()
        return 0
    lax.fori_loop(0, nrows // ZR, drain, 0)


def _load_idx(hbm2d, idx_v, row0, ngt):
    def body(k, _):
        pltpu.sync_copy(hbm2d.at[pl.ds(row0 + k * 8, 8)], idx_v.at[pl.ds(k * 8, 8)])
        return 0
    lax.fori_loop(0, ngt // 8, body, 0)


def _pipelined_agg(tab, src_v, dst_v, rows_v, acc_sh, sems, ngt):
    r = [rows_v.at[j] for j in range(3)]

    def gather(g, j):
        pltpu.async_copy(tab.at[src_v.at[g]], r[j], sems[j])

    def wait_g(j):
        pltpu.make_async_copy(tab.at[src_v.at[0]], r[j], sems[j]).wait()

    def scatter(g, j):
        pltpu.async_copy(r[j], acc_sh.at[dst_v.at[g]], sems[j], add=True)

    def wait_s(j):
        pltpu.make_async_copy(r[j], acc_sh.at[dst_v.at[0]], sems[j]).wait()

    gather(0, 0)
    gather(1, 1)
    gather(2, 2)
    wait_g(0)
    scatter(0, 0)

    def block(p, _):
        gbase = 1 + 3 * p
        for j3 in range(3):
            g = gbase + j3
            wait_s(j3)
            gather(jnp.minimum(g + 2, ngt - 1), j3)
            wait_g((1 + j3) % 3)
            scatter(g, (1 + j3) % 3)
        return 0

    lax.fori_loop(0, (ngt - 2) // 3, block, 0)

    wait_s(0)
    wait_g(1)
    scatter(ngt - 1, 1)
    wait_g(2)
    wait_s(1)


def _deg_scatter(ones_v, dst_v, acc_sh, ss0, ngt):
    def chunk(cc, _):
        for j in range(8):
            pltpu.async_copy(ones_v, acc_sh.at[dst_v.at[cc * 8 + j]], ss0, add=True)
        for j in range(8):
            pltpu.make_async_copy(ones_v, acc_sh.at[dst_v.at[cc * 8 + j]], ss0).wait()
        return 0

    lax.fori_loop(0, ngt // 8, chunk, 0)


def _mesh():
    return plsc.VectorSubcoreMesh(
        core_axis_name="c", subcore_axis_name="s", num_cores=NC, num_subcores=NS
    )


def _make_sc_agg(ng):
    @functools.partial(
        pl.kernel,
        out_type=jax.ShapeDtypeStruct((2, NC, N_PAD, D_IN), jnp.float32),
        mesh=_mesh(),
        scratch_types=(
            pltpu.VMEM_SHARED((N_PAD, D_IN), jnp.float32),
            pltpu.VMEM((ng, G), jnp.int32),
            pltpu.VMEM((ng, G), jnp.int32),
            pltpu.VMEM((3, G, D_IN), jnp.float32),
            pltpu.VMEM((ZR, D_IN), jnp.float32),
        ) + tuple([pltpu.SemaphoreType.DMA] * 3) + (
        ),
    )
    def sc_agg(t0_hbm, t1_hbm, s0_hbm, d0_hbm, s1_hbm, d1_hbm, agg_out,
               acc_sh, src_v, dst_v, rows_v, zbuf, m0, m1, m2):
        c = lax.axis_index("c")
        s = lax.axis_index("s")
        ngt = jnp.where(c < NC, ng, 0)
        row0 = (c * NS + s) * ngt
        base = s * ROWS_PER_TILE

        _fill_zero_rows(zbuf, ZR, jnp.float32)
        _zero_slice(zbuf, acc_sh, base, ROWS_PER_TILE, m0)
        plsc.subcore_barrier()

        for r, (tab, sh, dh) in enumerate(((t0_hbm, s0_hbm, d0_hbm),
                                           (t1_hbm, s1_hbm, d1_hbm))):
            _load_idx(sh, src_v, row0, ngt)
            _load_idx(dh, dst_v, row0, ngt)
            _pipelined_agg(tab, src_v, dst_v, rows_v, acc_sh,
                           (m0, m1, m2), ngt)
            plsc.subcore_barrier()
            pltpu.sync_copy(acc_sh.at[pl.ds(base, ROWS_PER_TILE)],
                            agg_out.at[r, c, pl.ds(base, ROWS_PER_TILE)])
            if r == 0:
                _zero_slice(zbuf, acc_sh, base, ROWS_PER_TILE, m0)
                plsc.subcore_barrier()

    return sc_agg


def _norms(deg):
    n0 = 1.0 / jnp.clip(deg[0, 0, :, 0] + deg[0, 1, :, 0], 1.0, None)
    n1 = 1.0 / jnp.clip(deg[1, 0, :, 0] + deg[1, 1, :, 0], 1.0, None)
    return n0, n1


def _tc1_body(agg_ref, deg_ref, w00_ref, w01_ref, b00_ref, b01_ref,
              w10_ref, w11_ref, m0_ref, m1_ref):
    agg = agg_ref[...]
    n0, n1 = _norms(deg_ref[...])
    a0 = (agg[0, 0] + agg[0, 1]) * n0[:, None]
    a1 = (agg[1, 0] + agg[1, 1]) * n1[:, None]
    h = jnp.dot(a0, w00_ref[...], preferred_element_type=jnp.float32)
    h += jnp.dot(a1, w01_ref[...], preferred_element_type=jnp.float32)
    h += b00_ref[...] + b01_ref[...]
    h = jnp.maximum(h, 0.0)
    m0_ref[...] = jnp.dot(h, w10_ref[...], preferred_element_type=jnp.float32)
    m1_ref[...] = jnp.dot(h, w11_ref[...], preferred_element_type=jnp.float32)


def _tc2_body(agg_ref, deg_ref, b10_ref, b11_ref, out_ref):
    agg = agg_ref[...]
    n0, n1 = _norms(deg_ref[...])
    out = (agg[0, 0] + agg[0, 1]) * n0[:, None]
    out += (agg[1, 0] + agg[1, 1]) * n1[:, None]
    out_ref[...] = out + b10_ref[...] + b11_ref[...]


def _tc1(agg0, deg, w00, w01, b00, b01, w10, w11):
    grid = (N_PAD // BN,)
    return pl.pallas_call(
        _tc1_body,
        grid=grid,
        in_specs=[
            pl.BlockSpec((2, NC, BN, D_IN), lambda i: (0, 0, i, 0)),
            pl.BlockSpec((2, NC, BN, D_IN), lambda i: (0, 0, i, 0)),
            pl.BlockSpec((D_IN, D_HID), lambda i: (0, 0)),
            pl.BlockSpec((D_IN, D_HID), lambda i: (0, 0)),
            pl.BlockSpec((1, D_HID), lambda i: (0, 0)),
            pl.BlockSpec((1, D_HID), lambda i: (0, 0)),
            pl.BlockSpec((D_HID, D_OUT), lambda i: (0, 0)),
            pl.BlockSpec((D_HID, D_OUT), lambda i: (0, 0)),
        ],
        out_specs=[
            pl.BlockSpec((BN, D_OUT), lambda i: (i, 0)),
            pl.BlockSpec((BN, D_OUT), lambda i: (i, 0)),
        ],
        out_shape=[
            jax.ShapeDtypeStruct((N_PAD, D_OUT), jnp.float32),
            jax.ShapeDtypeStruct((N_PAD, D_OUT), jnp.float32),
        ],
    )(agg0, deg, w00, w01, b00, b01, w10, w11)


def _tc2(agg1, deg, b10, b11):
    grid = (N_PAD // BN,)
    return pl.pallas_call(
        _tc2_body,
        grid=grid,
        in_specs=[
            pl.BlockSpec((2, NC, BN, D_OUT), lambda i: (0, 0, i, 0)),
            pl.BlockSpec((2, NC, BN, D_IN), lambda i: (0, 0, i, 0)),
            pl.BlockSpec((1, D_OUT), lambda i: (0, 0)),
            pl.BlockSpec((1, D_OUT), lambda i: (0, 0)),
        ],
        out_specs=pl.BlockSpec((BN, D_OUT), lambda i: (i, 0)),
        out_shape=jax.ShapeDtypeStruct((N_PAD, D_OUT), jnp.float32),
    )(agg1, deg, b10, b11)


def kernel(x, edge_index_r0, edge_index_r1, W0_r0, b0_r0, W0_r1, b0_r1,
           W1_r0, b1_r0, W1_r1, b1_r1):
    E = edge_index_r0.shape[1]
    gp_tile = -(-E // (NW * G * 8)) * 8
    while gp_tile % 3 != 2:
        gp_tile += 8
    e_pad = NW * G * gp_tile


    xp = jnp.zeros((N_PAD, D_IN), jnp.float32).at[:N_NODES].set(x)

    def prep(ei):
        idx = ei.astype(jnp.int32)
        pad = jnp.full((e_pad - E,), DUMMY, jnp.int32)
        s = jnp.concatenate([idx[0], pad]).reshape(e_pad // G, G)
        d = jnp.concatenate([idx[1], pad]).reshape(e_pad // G, G)
        return s, d

    s0, d0 = prep(edge_index_r0)
    s1, d1 = prep(edge_index_r1)

    agg0, deg = _make_sc_agg0deg(gp_tile)(xp, s0, d0, s1, d1)
    m0, m1 = _tc1(agg0, deg, W0_r0, W0_r1,
                  b0_r0.reshape(1, -1), b0_r1.reshape(1, -1), W1_r0, W1_r1)
    agg1 = _make_sc_agg(gp_tile)(m0, m1, s0, d0, s1, d1)
    out = _tc2(agg1, deg, b1_r0.reshape(1, -1), b1_r1.reshape(1, -1))
    return out[:N_NODES]


# speedup vs baseline: 1.4835x; 1.0001x over previous
import functools

import jax
import jax.numpy as jnp
from jax import lax
from jax.experimental import pallas as pl
from jax.experimental.pallas import tpu as pltpu
from jax.experimental.pallas import tpu_sc as plsc

N_NODES = 10000
D_IN = 128
D_HID = 256
D_OUT = 128

NC = 2
NS = 16
NW = NC * NS
G = 64
N_PAD = 10240
ROWS_PER_TILE = N_PAD // NS
DUMMY = N_NODES
BN = 512
ZR = 16


def _fill_zero_rows(ref, nrows, dtype):
    def body(i, _):
        for k in range(D_IN // 16):
            ref[i, pl.ds(k * 16, 16)] = jnp.zeros((16,), dtype)
        return 0
    lax.fori_loop(0, nrows, body, 0)


def _fill_one_rows(ref, nrows):
    def body(i, _):
        for k in range(D_IN // 16):
            ref[i, pl.ds(k * 16, 16)] = jnp.ones((16,), jnp.float32)
        return 0
    lax.fori_loop(0, nrows, body, 0)


def _make_sc_agg0deg(ng):
    @functools.partial(
        pl.kernel,
        out_type=(
            jax.ShapeDtypeStruct((2, NC, N_PAD, D_IN), jnp.float32),
            jax.ShapeDtypeStruct((2, NC, N_PAD, D_IN), jnp.float32),
        ),
        mesh=_mesh(),
        scratch_types=(
            pltpu.VMEM_SHARED((N_PAD, D_IN), jnp.float32),
            pltpu.VMEM((ng, G), jnp.int32),
            pltpu.VMEM((ng, G), jnp.int32),
            pltpu.VMEM((3, G, D_IN), jnp.float32),
            pltpu.VMEM((ZR, D_IN), jnp.float32),
        ) + tuple([pltpu.SemaphoreType.DMA] * 3) + (
        ),
    )
    def sc_agg0deg(x_hbm, s0_hbm, d0_hbm, s1_hbm, d1_hbm, agg_out, deg_out,
                   acc_sh, src_v, dst_v, rows_v, zbuf, m0, m1, m2):
        c = lax.axis_index("c")
        s = lax.axis_index("s")
        ngt = jnp.where(c < NC, ng, 0)
        row0 = (c * NS + s) * ngt
        base = s * ROWS_PER_TILE

        _fill_zero_rows(zbuf, ZR, jnp.float32)
        _zero_slice(zbuf, acc_sh, base, ROWS_PER_TILE, m0)
        plsc.subcore_barrier()

        for r, (sh, dh) in enumerate(((s0_hbm, d0_hbm), (s1_hbm, d1_hbm))):
            _load_idx(sh, src_v, row0, ngt)
            _load_idx(dh, dst_v, row0, ngt)
            _pipelined_agg(x_hbm, src_v, dst_v, rows_v, acc_sh,
                           (m0, m1, m2), ngt)
            plsc.subcore_barrier()
            pltpu.sync_copy(acc_sh.at[pl.ds(base, ROWS_PER_TILE)],
                            agg_out.at[r, c, pl.ds(base, ROWS_PER_TILE)])
            _zero_slice(zbuf, acc_sh, base, ROWS_PER_TILE, m0)
            plsc.subcore_barrier()

        ones_v = rows_v.at[0]
        _fill_one_rows(ones_v, G)
        for r, dh in enumerate((d0_hbm, d1_hbm)):
            _load_idx(dh, dst_v, row0, ngt)
            _deg_scatter(ones_v, dst_v, acc_sh, m1, ngt)
            plsc.subcore_barrier()
            pltpu.sync_copy(acc_sh.at[pl.ds(base, ROWS_PER_TILE)],
                            deg_out.at[r, c, pl.ds(base, ROWS_PER_TILE)])
            if r == 0:
                _zero_slice(zbuf, acc_sh, base, ROWS_PER_TILE, m0)
                plsc.subcore_barrier()

    return sc_agg0deg


def _zero_slice(zbuf, dst_sh, base, nrows, sem):
    def issue(k, _):
        pltpu.async_copy(zbuf, dst_sh.at[pl.ds(base + k * ZR, ZR)], sem)
        return 0
    lax.fori_loop(0, nrows // ZR, issue, 0)

    def drain(k, _):
        pltpu.make_async_copy(zbuf, dst_sh.at[pl.ds(base, ZR)], sem).wait()
        return 0
    lax.fori_loop(0, nrows // ZR, drain, 0)


def _load_idx(hbm2d, idx_v, row0, ngt):
    def body(k, _):
        pltpu.sync_copy(hbm2d.at[pl.ds(row0 + k * 8, 8)], idx_v.at[pl.ds(k * 8, 8)])
        return 0
    lax.fori_loop(0, ngt // 8, body, 0)


def _pipelined_agg(tab, src_v, dst_v, rows_v, acc_sh, sems, ngt):
    r = [rows_v.at[j] for j in range(3)]

    def gather(g, j):
        pltpu.async_copy(tab.at[src_v.at[g]], r[j], sems[j])

    def wait_g(j):
        pltpu.make_async_copy(tab.at[src_v.at[0]], r[j], sems[j]).wait()

    def scatter(g, j):
        pltpu.async_copy(r[j], acc_sh.at[dst_v.at[g]], sems[j], add=True)

    def wait_s(j):
        pltpu.make_async_copy(r[j], acc_sh.at[dst_v.at[0]], sems[j]).wait()

    gather(0, 0)
    gather(1, 1)
    gather(2, 2)
    wait_g(0)
    scatter(0, 0)

    def block(p, _):
        gbase = 1 + 3 * p
        for j3 in range(3):
            g = gbase + j3
            wait_s(j3)
            gather(jnp.minimum(g + 2, ngt - 1), j3)
            wait_g((1 + j3) % 3)
            scatter(g, (1 + j3) % 3)
        return 0

    lax.fori_loop(0, (ngt - 2) // 3, block, 0)

    wait_s(0)
    wait_g(1)
    scatter(ngt - 1, 1)
    wait_g(2)
    wait_s(1)


def _deg_scatter(ones_v, dst_v, acc_sh, ss0, ngt):
    def chunk(cc, _):
        for j in range(8):
            pltpu.async_copy(ones_v, acc_sh.at[dst_v.at[cc * 8 + j]], ss0, add=True)
        for j in range(8):
            pltpu.make_async_copy(ones_v, acc_sh.at[dst_v.at[cc * 8 + j]], ss0).wait()
        return 0

    lax.fori_loop(0, ngt // 8, chunk, 0)


def _mesh():
    return plsc.VectorSubcoreMesh(
        core_axis_name="c", subcore_axis_name="s", num_cores=NC, num_subcores=NS
    )


def _make_sc_agg(ng):
    @functools.partial(
        pl.kernel,
        out_type=jax.ShapeDtypeStruct((2, NC, N_PAD, D_IN), jnp.float32),
        mesh=_mesh(),
        scratch_types=(
            pltpu.VMEM_SHARED((N_PAD, D_IN), jnp.float32),
            pltpu.VMEM((ng, G), jnp.int32),
            pltpu.VMEM((ng, G), jnp.int32),
            pltpu.VMEM((3, G, D_IN), jnp.float32),
            pltpu.VMEM((ZR, D_IN), jnp.float32),
        ) + tuple([pltpu.SemaphoreType.DMA] * 3) + (
        ),
    )
    def sc_agg(t0_hbm, t1_hbm, s0_hbm, d0_hbm, s1_hbm, d1_hbm, agg_out,
               acc_sh, src_v, dst_v, rows_v, zbuf, m0, m1, m2):
        c = lax.axis_index("c")
        s = lax.axis_index("s")
        ngt = jnp.where(c < NC, ng, 0)
        row0 = (c * NS + s) * ngt
        base = s * ROWS_PER_TILE

        _fill_zero_rows(zbuf, ZR, jnp.float32)
        _zero_slice(zbuf, acc_sh, base, ROWS_PER_TILE, m0)
        plsc.subcore_barrier()

        for r, (tab, sh, dh) in enumerate(((t0_hbm, s0_hbm, d0_hbm),
                                           (t1_hbm, s1_hbm, d1_hbm))):
            _load_idx(sh, src_v, row0, ngt)
            _load_idx(dh, dst_v, row0, ngt)
            _pipelined_agg(tab, src_v, dst_v, rows_v, acc_sh,
                           (m0, m1, m2), ngt)
            plsc.subcore_barrier()
            pltpu.sync_copy(acc_sh.at[pl.ds(base, ROWS_PER_TILE)],
                            agg_out.at[r, c, pl.ds(base, ROWS_PER_TILE)])
            if r == 0:
                _zero_slice(zbuf, acc_sh, base, ROWS_PER_TILE, m0)
                plsc.subcore_barrier()

    return sc_agg


def _norms(deg):
    n0 = 1.0 / jnp.clip(deg[0, 0, :, 0] + deg[0, 1, :, 0], 1.0, None)
    n1 = 1.0 / jnp.clip(deg[1, 0, :, 0] + deg[1, 1, :, 0], 1.0, None)
    return n0, n1


def _tc1_body(agg_ref, deg_ref, w00_ref, w01_ref, b00_ref, b01_ref,
              w10_ref, w11_ref, m0_ref, m1_ref):
    agg = agg_ref[...]
    n0, n1 = _norms(deg_ref[...])
    a0 = (agg[0, 0] + agg[0, 1]) * n0[:, None]
    a1 = (agg[1, 0] + agg[1, 1]) * n1[:, None]
    h = jnp.dot(a0, w00_ref[...], preferred_element_type=jnp.float32)
    h += jnp.dot(a1, w01_ref[...], preferred_element_type=jnp.float32)
    h += b00_ref[...] + b01_ref[...]
    h = jnp.maximum(h, 0.0)
    m0_ref[...] = jnp.dot(h, w10_ref[...], preferred_element_type=jnp.float32)
    m1_ref[...] = jnp.dot(h, w11_ref[...], preferred_element_type=jnp.float32)


def _tc2_body(agg_ref, deg_ref, b10_ref, b11_ref, out_ref):
    agg = agg_ref[...]
    n0, n1 = _norms(deg_ref[...])
    out = (agg[0, 0] + agg[0, 1]) * n0[:, None]
    out += (agg[1, 0] + agg[1, 1]) * n1[:, None]
    out_ref[...] = out + b10_ref[...] + b11_ref[...]


def _tc1(agg0, deg, w00, w01, b00, b01, w10, w11):
    grid = (N_PAD // BN,)
    return pl.pallas_call(
        _tc1_body,
        grid=grid,
        in_specs=[
            pl.BlockSpec((2, NC, BN, D_IN), lambda i: (0, 0, i, 0)),
            pl.BlockSpec((2, NC, BN, D_IN), lambda i: (0, 0, i, 0)),
            pl.BlockSpec((D_IN, D_HID), lambda i: (0, 0)),
            pl.BlockSpec((D_IN, D_HID), lambda i: (0, 0)),
            pl.BlockSpec((1, D_HID), lambda i: (0, 0)),
            pl.BlockSpec((1, D_HID), lambda i: (0, 0)),
            pl.BlockSpec((D_HID, D_OUT), lambda i: (0, 0)),
            pl.BlockSpec((D_HID, D_OUT), lambda i: (0, 0)),
        ],
        out_specs=[
            pl.BlockSpec((BN, D_OUT), lambda i: (i, 0)),
            pl.BlockSpec((BN, D_OUT), lambda i: (i, 0)),
        ],
        out_shape=[
            jax.ShapeDtypeStruct((N_PAD, D_OUT), jnp.float32),
            jax.ShapeDtypeStruct((N_PAD, D_OUT), jnp.float32),
        ],
    )(agg0, deg, w00, w01, b00, b01, w10, w11)


def _tc2(agg1, deg, b10, b11):
    grid = (N_PAD // BN,)
    return pl.pallas_call(
        _tc2_body,
        grid=grid,
        in_specs=[
            pl.BlockSpec((2, NC, BN, D_OUT), lambda i: (0, 0, i, 0)),
            pl.BlockSpec((2, NC, BN, D_IN), lambda i: (0, 0, i, 0)),
            pl.BlockSpec((1, D_OUT), lambda i: (0, 0)),
            pl.BlockSpec((1, D_OUT), lambda i: (0, 0)),
        ],
        out_specs=pl.BlockSpec((BN, D_OUT), lambda i: (i, 0)),
        out_shape=jax.ShapeDtypeStruct((N_PAD, D_OUT), jnp.float32),
    )(agg1, deg, b10, b11)


def kernel(x, edge_index_r0, edge_index_r1, W0_r0, b0_r0, W0_r1, b0_r1,
           W1_r0, b1_r0, W1_r1, b1_r1):
    E = edge_index_r0.shape[1]
    gp_tile = -(-E // (NW * G * 8)) * 8
    while gp_tile % 3 != 2:
        gp_tile += 8
    e_pad = NW * G * gp_tile


    xp = jnp.zeros((N_PAD, D_IN), jnp.float32).at[:N_NODES].set(x)

    def prep(ei):
        idx = ei.astype(jnp.int32)
        pad = jnp.full((e_pad - E,), DUMMY, jnp.int32)
        s = jnp.concatenate([idx[0], pad]).reshape(e_pad // G, G)
        d = jnp.concatenate([idx[1], pad]).reshape(e_pad // G, G)
        return s, d

    s0, d0 = prep(edge_index_r0)
    s1, d1 = prep(edge_index_r1)

    agg0, deg = _make_sc_agg0deg(gp_tile)(xp, s0, d0, s1, d1)
    m0, m1 = _tc1(agg0, deg, W0_r0, W0_r1,
                  b0_r0.reshape(1, -1), b0_r1.reshape(1, -1), W1_r0, W1_r1)
    agg1 = _make_sc_agg(gp_tile)(m0, m1, s0, d0, s1, d1)
    out = _tc2(agg1, deg, b1_r0.reshape(1, -1), b1_r1.reshape(1, -1))
    return out[:N_NODES]


# speedup vs baseline: 1.4852x; 1.0012x over previous
import functools

import jax
import jax.numpy as jnp
from jax import lax
from jax.experimental import pallas as pl
from jax.experimental.pallas import tpu as pltpu
from jax.experimental.pallas import tpu_sc as plsc

N_NODES = 10000
D_IN = 128
D_HID = 256
D_OUT = 128

NC = 2
NS = 16
NW = NC * NS
G = 64
N_PAD = 10240
ROWS_PER_TILE = N_PAD // NS
DUMMY = N_NODES
BN = 512
ZR = 16


def _fill_zero_rows(ref, nrows, dtype):
    def body(i, _):
        for k in range(D_IN // 16):
            ref[i, pl.ds(k * 16, 16)] = jnp.zeros((16,), dtype)
        return 0
    lax.fori_loop(0, nrows, body, 0)


def _fill_one_rows(ref, nrows):
    def body(i, _):
        for k in range(D_IN // 16):
            ref[i, pl.ds(k * 16, 16)] = jnp.ones((16,), jnp.float32)
        return 0
    lax.fori_loop(0, nrows, body, 0)


def _make_sc_agg0deg(ng):
    @functools.partial(
        pl.kernel,
        out_type=(
            jax.ShapeDtypeStruct((2, NC, N_PAD, D_IN), jnp.float32),
            jax.ShapeDtypeStruct((2, NC, N_PAD, D_IN), jnp.float32),
        ),
        mesh=_mesh(),
        scratch_types=(
            pltpu.VMEM_SHARED((N_PAD, D_IN), jnp.float32),
            pltpu.VMEM((ng, G), jnp.int32),
            pltpu.VMEM((ng, G), jnp.int32),
            pltpu.VMEM((3, G, D_IN), jnp.float32),
            pltpu.VMEM((ZR, D_IN), jnp.float32),
        ) + tuple([pltpu.SemaphoreType.DMA] * 3) + (
        ),
    )
    def sc_agg0deg(x_hbm, s0_hbm, d0_hbm, s1_hbm, d1_hbm, agg_out, deg_out,
                   acc_sh, src_v, dst_v, rows_v, zbuf, m0, m1, m2):
        c = lax.axis_index("c")
        s = lax.axis_index("s")
        ngt = jnp.where(c < NC, ng, 0)
        row0 = (c * NS + s) * ngt
        base = s * ROWS_PER_TILE

        _fill_zero_rows(zbuf, ZR, jnp.float32)
        _zero_slice(zbuf, acc_sh, base, ROWS_PER_TILE, m0)
        plsc.subcore_barrier()

        for r, (sh, dh) in enumerate(((s0_hbm, d0_hbm), (s1_hbm, d1_hbm))):
            _load_idx(sh, src_v, row0, ngt)
            _load_idx(dh, dst_v, row0, ngt)
            _pipelined_agg(x_hbm, src_v, dst_v, rows_v, acc_sh,
                           (m0, m1, m2), ngt)
            plsc.subcore_barrier()
            pltpu.sync_copy(acc_sh.at[pl.ds(base, ROWS_PER_TILE)],
                            agg_out.at[r, c, pl.ds(base, ROWS_PER_TILE)])
            _zero_slice(zbuf, acc_sh, base, ROWS_PER_TILE, m0)
            plsc.subcore_barrier()

        ones_v = rows_v.at[0]
        _fill_one_rows(ones_v, G)
        for r, dh in enumerate((d0_hbm, d1_hbm)):
            _load_idx(dh, dst_v, row0, ngt)
            _deg_scatter(ones_v, dst_v, acc_sh, m1, ngt)
            plsc.subcore_barrier()
            pltpu.sync_copy(acc_sh.at[pl.ds(base, ROWS_PER_TILE)],
                            deg_out.at[r, c, pl.ds(base, ROWS_PER_TILE)])
            if r == 0:
                _zero_slice(zbuf, acc_sh, base, ROWS_PER_TILE, m0)
                plsc.subcore_barrier()

    return sc_agg0deg


def _zero_slice(zbuf, dst_sh, base, nrows, sem):
    def issue(k, _):
        pltpu.async_copy(zbuf, dst_sh.at[pl.ds(base + k * ZR, ZR)], sem)
        return 0
    lax.fori_loop(0, nrows // ZR, issue, 0)

    def drain(k, _):
        pltpu.make_async_copy(zbuf, dst_sh.at[pl.ds(base, ZR)], sem).wait()
        return 0
    lax.fori_loop(0, nrows // ZR, drain, 0)


def _load_idx(hbm2d, idx_v, row0, ngt):
    def body(k, _):
        pltpu.sync_copy(hbm2d.at[pl.ds(row0 + k * 8, 8)], idx_v.at[pl.ds(k * 8, 8)])
        return 0
    lax.fori_loop(0, ngt // 8, body, 0)


def _pipelined_agg(tab, src_v, dst_v, rows_v, acc_sh, sems, ngt):
    r = [rows_v.at[j] for j in range(3)]

    def gather(g, j):
        pltpu.async_copy(tab.at[src_v.at[g]], r[j], sems[j])

    def wait_g(j):
        pltpu.make_async_copy(tab.at[src_v.at[0]], r[j], sems[j]).wait()

    def scatter(g, j):
        pltpu.async_copy(r[j], acc_sh.at[dst_v.at[g]], sems[j], add=True)

    def wait_s(j):
        pltpu.make_async_copy(r[j], acc_sh.at[dst_v.at[0]], sems[j]).wait()

    gather(0, 0)
    gather(1, 1)
    gather(2, 2)
    wait_g(0)
    scatter(0, 0)

    def block(p, _):
        gbase = 1 + 3 * p
        for j3 in range(3):
            g = gbase + j3
            wait_s(j3)
            gather(jnp.minimum(g + 2, ngt - 1), j3)
            wait_g((1 + j3) % 3)
            scatter(g, (1 + j3) % 3)
        return 0

    lax.fori_loop(0, (ngt - 2) // 3, block, 0)

    wait_s(0)
    wait_g(1)
    scatter(ngt - 1, 1)
    wait_g(2)
    wait_s(1)


def _deg_scatter(ones_v, dst_v, acc_sh, ss0, ngt):
    def chunk(cc, _):
        for j in range(8):
            pltpu.async_copy(ones_v, acc_sh.at[dst_v.at[cc * 8 + j]], ss0, add=True)
        for j in range(8):
            pltpu.make_async_copy(ones_v, acc_sh.at[dst_v.at[cc * 8 + j]], ss0).wait()
        return 0

    lax.fori_loop(0, ngt // 8, chunk, 0)


def _mesh():
    return plsc.VectorSubcoreMesh(
        core_axis_name="c", subcore_axis_name="s", num_cores=NC, num_subcores=NS
    )


def _make_sc_agg(ng):
    @functools.partial(
        pl.kernel,
        out_type=jax.ShapeDtypeStruct((2, NC, N_PAD, D_IN), jnp.float32),
        mesh=_mesh(),
        scratch_types=(
            pltpu.VMEM_SHARED((N_PAD, D_IN), jnp.float32),
            pltpu.VMEM((ng, G), jnp.int32),
            pltpu.VMEM((ng, G), jnp.int32),
            pltpu.VMEM((3, G, D_IN), jnp.float32),
            pltpu.VMEM((ZR, D_IN), jnp.float32),
        ) + tuple([pltpu.SemaphoreType.DMA] * 3) + (
        ),
    )
    def sc_agg(t0_hbm, t1_hbm, s0_hbm, d0_hbm, s1_hbm, d1_hbm, agg_out,
               acc_sh, src_v, dst_v, rows_v, zbuf, m0, m1, m2):
        c = lax.axis_index("c")
        s = lax.axis_index("s")
        ngt = jnp.where(c < NC, ng, 0)
        row0 = (c * NS + s) * ngt
        base = s * ROWS_PER_TILE

        _fill_zero_rows(zbuf, ZR, jnp.float32)
        _zero_slice(zbuf, acc_sh, base, ROWS_PER_TILE, m0)
        plsc.subcore_barrier()

        for r, (tab, sh, dh) in enumerate(((t0_hbm, s0_hbm, d0_hbm),
                                           (t1_hbm, s1_hbm, d1_hbm))):
            _load_idx(sh, src_v, row0, ngt)
            _load_idx(dh, dst_v, row0, ngt)
            _pipelined_agg(tab, src_v, dst_v, rows_v, acc_sh,
                           (m0, m1, m2), ngt)
            plsc.subcore_barrier()
            pltpu.sync_copy(acc_sh.at[pl.ds(base, ROWS_PER_TILE)],
                            agg_out.at[r, c, pl.ds(base, ROWS_PER_TILE)])
            if r == 0:
                _zero_slice(zbuf, acc_sh, base, ROWS_PER_TILE, m0)
                plsc.subcore_barrier()

    return sc_agg


def _norms(deg):
    n0 = 1.0 / jnp.clip(deg[0, 0, :, 0] + deg[0, 1, :, 0], 1.0, None)
    n1 = 1.0 / jnp.clip(deg[1, 0, :, 0] + deg[1, 1, :, 0], 1.0, None)
    return n0, n1


def _tc1_body(agg_ref, deg_ref, w00_ref, w01_ref, b00_ref, b01_ref,
              w10_ref, w11_ref, m0_ref, m1_ref, nrm_ref):
    agg = agg_ref[...]
    n0, n1 = _norms(deg_ref[...])
    nrm_ref[0] = n0
    nrm_ref[1] = n1
    a0 = (agg[0, 0] + agg[0, 1]) * n0[:, None]
    a1 = (agg[1, 0] + agg[1, 1]) * n1[:, None]
    h = jnp.dot(a0, w00_ref[...], preferred_element_type=jnp.float32)
    h += jnp.dot(a1, w01_ref[...], preferred_element_type=jnp.float32)
    h += b00_ref[...] + b01_ref[...]
    h = jnp.maximum(h, 0.0)
    m0_ref[...] = jnp.dot(h, w10_ref[...], preferred_element_type=jnp.float32)
    m1_ref[...] = jnp.dot(h, w11_ref[...], preferred_element_type=jnp.float32)


def _tc2_body(agg_ref, nrm_ref, b10_ref, b11_ref, out_ref):
    agg = agg_ref[...]
    n0 = nrm_ref[0]
    n1 = nrm_ref[1]
    out = (agg[0, 0] + agg[0, 1]) * n0[:, None]
    out += (agg[1, 0] + agg[1, 1]) * n1[:, None]
    out_ref[...] = out + b10_ref[...] + b11_ref[...]


def _tc1(agg0, deg, w00, w01, b00, b01, w10, w11):
    grid = (N_PAD // BN,)
    return pl.pallas_call(
        _tc1_body,
        grid=grid,
        in_specs=[
            pl.BlockSpec((2, NC, BN, D_IN), lambda i: (0, 0, i, 0)),
            pl.BlockSpec((2, NC, BN, D_IN), lambda i: (0, 0, i, 0)),
            pl.BlockSpec((D_IN, D_HID), lambda i: (0, 0)),
            pl.BlockSpec((D_IN, D_HID), lambda i: (0, 0)),
            pl.BlockSpec((1, D_HID), lambda i: (0, 0)),
            pl.BlockSpec((1, D_HID), lambda i: (0, 0)),
            pl.BlockSpec((D_HID, D_OUT), lambda i: (0, 0)),
            pl.BlockSpec((D_HID, D_OUT), lambda i: (0, 0)),
        ],
        out_specs=[
            pl.BlockSpec((BN, D_OUT), lambda i: (i, 0)),
            pl.BlockSpec((BN, D_OUT), lambda i: (i, 0)),
            pl.BlockSpec((2, BN), lambda i: (0, i)),
        ],
        out_shape=[
            jax.ShapeDtypeStruct((N_PAD, D_OUT), jnp.float32),
            jax.ShapeDtypeStruct((N_PAD, D_OUT), jnp.float32),
            jax.ShapeDtypeStruct((2, N_PAD), jnp.float32),
        ],
    )(agg0, deg, w00, w01, b00, b01, w10, w11)


def _tc2(agg1, nrm, b10, b11):
    grid = (N_PAD // BN,)
    return pl.pallas_call(
        _tc2_body,
        grid=grid,
        in_specs=[
            pl.BlockSpec((2, NC, BN, D_OUT), lambda i: (0, 0, i, 0)),
            pl.BlockSpec((2, BN), lambda i: (0, i)),
            pl.BlockSpec((1, D_OUT), lambda i: (0, 0)),
            pl.BlockSpec((1, D_OUT), lambda i: (0, 0)),
        ],
        out_specs=pl.BlockSpec((BN, D_OUT), lambda i: (i, 0)),
        out_shape=jax.ShapeDtypeStruct((N_PAD, D_OUT), jnp.float32),
    )(agg1, nrm, b10, b11)


def kernel(x, edge_index_r0, edge_index_r1, W0_r0, b0_r0, W0_r1, b0_r1,
           W1_r0, b1_r0, W1_r1, b1_r1):
    E = edge_index_r0.shape[1]
    gp_tile = -(-E // (NW * G * 8)) * 8
    while gp_tile % 3 != 2:
        gp_tile += 8
    e_pad = NW * G * gp_tile


    xp = jnp.zeros((N_PAD, D_IN), jnp.float32).at[:N_NODES].set(x)

    def prep(ei):
        idx = ei.astype(jnp.int32)
        pad = jnp.full((e_pad - E,), DUMMY, jnp.int32)
        s = jnp.concatenate([idx[0], pad]).reshape(e_pad // G, G)
        d = jnp.concatenate([idx[1], pad]).reshape(e_pad // G, G)
        return s, d

    s0, d0 = prep(edge_index_r0)
    s1, d1 = prep(edge_index_r1)

    agg0, deg = _make_sc_agg0deg(gp_tile)(xp, s0, d0, s1, d1)
    m0, m1, nrm = _tc1(agg0, deg, W0_r0, W0_r1,
                       b0_r0.reshape(1, -1), b0_r1.reshape(1, -1), W1_r0, W1_r1)
    agg1 = _make_sc_agg(gp_tile)(m0, m1, s0, d0, s1, d1)
    out = _tc2(agg1, nrm, b1_r0.reshape(1, -1), b1_r1.reshape(1, -1))
    return out[:N_NODES]
